# Initial kernel scaffold; baseline (speedup 1.0000x reference)
import jax
import jax.numpy as jnp
from jax.experimental import pallas as pl


def kernel(x, edge_attr, edge_index, center_id, ee_W1, ee_b1, ee_W2, ee_b2, g0_eW, g0_eb, g0_W1, g0_b1, g0_W2, g0_b2, g1_eW, g1_eb, g1_W1, g1_b1, g1_W2, g1_b2, g2_eW, g2_eb, g2_W1, g2_b1, g2_W2, g2_b2, gat_Wl, gat_bl, gat_Wr, gat_br, gat_We, gat_att, gat_b, mu_W, mu_b, lv_W, lv_b, ln0_g, ln0_b, ln1_g, ln1_b, ln2_g, ln2_b, ln3_g, ln3_b):
    raise NotImplementedError("write your pallas kernel here")


# speedup vs baseline: 4.9831x; 4.9831x over previous
import functools

import jax
import jax.numpy as jnp
from jax import lax
from jax.experimental import pallas as pl
from jax.experimental.pallas import tpu as pltpu
from jax.experimental.pallas import tpu_sc as plsc

N = 10000
E = 320000
F_IN = 128
F_E = 16
HID = 384
LAT = 64
NH = 8
HC = 48
NHP = 16
F32 = jnp.float32

NC = 2
NS = 16
NW = NC * NS
NQ = 4
QH = HID // NQ
Q0 = F_IN // NQ

ZB = 80
NBN = N // ZB


def _edge_encode_project(ea, W1, b1, W2, b2, Wcat, bcat):
    Be = 512
    nb = E // Be

    def body(ea_ref, w1, bb1, w2, bb2, wc, bc, *outs):
        u = jnp.maximum(jnp.dot(ea_ref[...], w1[...],
                                preferred_element_type=F32) + bb1[...], 0.0)
        e = jnp.dot(u, w2[...], preferred_element_type=F32) + bb2[...]
        proj = jnp.dot(e, wc[...], preferred_element_type=F32) + bc[...]
        for q in range(NQ):
            outs[q][...] = proj[:, q * Q0:(q + 1) * Q0]
        for k in range(3 * NQ):
            outs[NQ + k][...] = proj[:, F_IN + k * QH:F_IN + (k + 1) * QH]

    full = lambda r, c: pl.BlockSpec((r, c), lambda i: (0, 0))
    eb = lambda c: pl.BlockSpec((Be, c), lambda i: (i, 0))
    sh = lambda c: jax.ShapeDtypeStruct((E, c), F32)
    return pl.pallas_call(
        body,
        grid=(nb,),
        in_specs=[
            pl.BlockSpec((Be, F_E), lambda i: (i, 0)),
            full(F_E, HID), full(1, HID), full(HID, HID), full(1, HID),
            full(HID, F_IN + 3 * HID), full(1, F_IN + 3 * HID),
        ],
        out_specs=[eb(Q0)] * NQ + [eb(QH)] * (3 * NQ),
        out_shape=[sh(Q0)] * NQ + [sh(QH)] * (3 * NQ),
    )(ea, W1, b1, W2, b2, Wcat, bcat)


def _gine_node(hin, agg4, res, W1, b1, W2, b2, g, b, has_res):
    Din = hin.shape[1]
    Dq = Din // NQ
    Bn = 1000
    nb = N // Bn
    nblk = N // Bn

    def body(*refs):
        if has_res:
            (hin_r, a0, a1, a2, a3, res_r, w1, bb1, w2, bb2, gg, bb,
             out) = refs
        else:
            (hin_r, a0, a1, a2, a3, w1, bb1, w2, bb2, gg, bb, out) = refs
        aggr = jnp.concatenate([a0[...], a1[...], a2[...], a3[...]], axis=1)
        z = hin_r[...] + aggr
        u = jnp.maximum(jnp.dot(z, w1[...], preferred_element_type=F32)
                        + bb1[...], 0.0)
        t = jnp.dot(u, w2[...], preferred_element_type=F32) + bb2[...]
        if has_res:
            t = t + res_r[...]
        m = jnp.mean(t, axis=1, keepdims=True)
        d = t - m
        v = jnp.mean(d * d, axis=1, keepdims=True)
        out[...] = d / jnp.sqrt(v + 1e-5) * gg[...] + bb[...]

    full = lambda r, c: pl.BlockSpec((r, c), lambda i: (0, 0))
    aview = lambda q: pl.BlockSpec(
        (Bn, Dq), lambda i, q=q: (i + q * nblk, 0))
    in_specs = [pl.BlockSpec((Bn, Din), lambda i: (i, 0)),
                aview(0), aview(1), aview(2), aview(3)]
    args = [hin, agg4, agg4, agg4, agg4]
    if has_res:
        in_specs.append(pl.BlockSpec((Bn, HID), lambda i: (i, 0)))
        args.append(res)
    in_specs += [full(Din, HID), full(1, HID), full(HID, HID), full(1, HID),
                 full(1, HID), full(1, HID)]
    args += [W1, b1, W2, b2, g, b]
    return pl.pallas_call(
        body,
        grid=(nb,),
        in_specs=in_specs,
        out_specs=pl.BlockSpec((Bn, HID), lambda i: (i, 0)),
        out_shape=jax.ShapeDtypeStruct((N, HID), F32),
    )(*args)


def _gat_proj(h, Wl, bl, Wr, br):
    Bn = 1000
    nb = N // Bn

    def body(h_r, wl, bbl, wr, bbr, oxl, oxr):
        hv = h_r[...]
        oxl[...] = jnp.dot(hv, wl[...], preferred_element_type=F32) + bbl[...]
        oxr[...] = jnp.dot(hv, wr[...], preferred_element_type=F32) + bbr[...]

    full = lambda r, c: pl.BlockSpec((r, c), lambda i: (0, 0))
    return pl.pallas_call(
        body,
        grid=(nb,),
        in_specs=[pl.BlockSpec((Bn, HID), lambda i: (i, 0)),
                  full(HID, HID), full(1, HID), full(HID, HID), full(1, HID)],
        out_specs=[pl.BlockSpec((Bn, HID), lambda i: (i, 0)),
                   pl.BlockSpec((Bn, HID), lambda i: (i, 0))],
        out_shape=[jax.ShapeDtypeStruct((N, HID), F32),
                   jax.ShapeDtypeStruct((N, HID), F32)],
    )(h, Wl, bl, Wr, br)


def _gat_edge(xlg, xrg, epgq, A, R):
    Be = 512
    nb = E // Be

    def body(xl_r, xr_r, e0, e1, e2, e3, a_r, r_r, oex, *ows):
        xlv = xl_r[...]
        epv = jnp.concatenate([e0[...], e1[...], e2[...], e3[...]], axis=1)
        q = xlv + xr_r[...] + epv
        ql = jnp.where(q >= 0.0, q, 0.2 * q)
        al = jnp.dot(ql, a_r[...], preferred_element_type=F32)
        ex = jnp.exp(al)
        oex[...] = ex
        w = xlv * jnp.dot(ex, r_r[...], preferred_element_type=F32)
        for k in range(NQ):
            ows[k][...] = w[:, k * QH:(k + 1) * QH]

    full = lambda r, c: pl.BlockSpec((r, c), lambda i: (0, 0))
    eb = lambda c: pl.BlockSpec((Be, c), lambda i: (i, 0))
    return pl.pallas_call(
        body,
        grid=(nb,),
        in_specs=[eb(HID), eb(HID)] + [eb(QH)] * NQ
        + [full(HID, NHP), full(NHP, HID)],
        out_specs=[eb(NHP)] + [eb(QH)] * NQ,
        out_shape=[jax.ShapeDtypeStruct((E, NHP), F32)]
        + [jax.ShapeDtypeStruct((E, QH), F32)] * NQ,
    )(xlg, xrg, *epgq, A, R)


def _gat_node(h, xl, xr, aggrg4, numer4, den2, A, R, gat_b, g, b):
    Bn = 1000
    nb = N // Bn
    nblk = N // Bn

    def body(h_r, xl_r, xr_r, ag0, ag1, ag2, ag3, nm0, nm1, nm2, nm3,
             d0, d1, a_r, r_r, gb, gg, bb, out):
        xlv = xl_r[...]
        den_sc = d0[...] + d1[...]
        deg = jnp.maximum(den_sc[:, 8:9], 1.0)
        lep = jnp.concatenate([ag0[...], ag1[...], ag2[...], ag3[...]],
                              axis=1) / deg
        q = xlv + xr_r[...] + lep
        ql = jnp.where(q >= 0.0, q, 0.2 * q)
        exl = jnp.exp(jnp.dot(ql, a_r[...], preferred_element_type=F32))
        den_t = den_sc + exl + 1e-16
        numer = jnp.concatenate([nm0[...], nm1[...], nm2[...], nm3[...]],
                                axis=1) \
            + xlv * jnp.dot(exl, r_r[...], preferred_element_type=F32)
        den_rep = jnp.dot(den_t, r_r[...], preferred_element_type=F32)
        gat = numer / den_rep + gb[...]
        t = h_r[...] + gat
        m = jnp.mean(t, axis=1, keepdims=True)
        d = t - m
        v = jnp.mean(d * d, axis=1, keepdims=True)
        out[...] = d / jnp.sqrt(v + 1e-5) * gg[...] + bb[...]

    full = lambda r, c: pl.BlockSpec((r, c), lambda i: (0, 0))
    qv = lambda q: pl.BlockSpec((Bn, QH), lambda i, q=q: (i + q * nblk, 0))
    hv = lambda: pl.BlockSpec((Bn, HID), lambda i: (i, 0))
    return pl.pallas_call(
        body,
        grid=(nb,),
        in_specs=[hv(), hv(), hv(), qv(0), qv(1), qv(2), qv(3),
                  qv(0), qv(1), qv(2), qv(3),
                  pl.BlockSpec((Bn, NHP), lambda i: (i, 0)),
                  pl.BlockSpec((Bn, NHP), lambda i: (i + nblk, 0)),
                  full(HID, NHP), full(NHP, HID), full(1, HID),
                  full(1, HID), full(1, HID)],
        out_specs=pl.BlockSpec((Bn, HID), lambda i: (i, 0)),
        out_shape=jax.ShapeDtypeStruct((N, HID), F32),
    )(h, xl, xr, aggrg4, aggrg4, aggrg4, aggrg4,
      numer4, numer4, numer4, numer4, den2, den2,
      A, R, gat_b, g, b)


def _heads(cv, mu_W, mu_b, lv_W, lv_b):
    def body(cv_r, wm, bm, wv, bv, omu, olv):
        c = cv_r[...]
        omu[...] = jnp.dot(c, wm[...], preferred_element_type=F32) + bm[...]
        olv[...] = jnp.dot(c, wv[...], preferred_element_type=F32) + bv[...]

    full = lambda r, c: pl.BlockSpec((r, c), lambda i: (0, 0))
    return pl.pallas_call(
        body,
        grid=(1,),
        in_specs=[pl.BlockSpec((256, HID), lambda i: (0, 0)),
                  full(HID, LAT), full(1, LAT), full(HID, LAT), full(1, LAT)],
        out_specs=[pl.BlockSpec((256, LAT), lambda i: (0, 0)),
                   pl.BlockSpec((256, LAT), lambda i: (0, 0))],
        out_shape=[jax.ShapeDtypeStruct((256, LAT), F32),
                   jax.ShapeDtypeStruct((256, LAT), F32)],
    )(cv, mu_W, mu_b, lv_W, lv_b)


_MESH = plsc.VectorSubcoreMesh(core_axis_name="c", subcore_axis_name="s")


def _acc_rows(s, fn):
    def body(k, _):
        blk = s + k * NS

        @pl.when(blk < NBN)
        def _():
            fn(blk * ZB)
        return 0

    lax.fori_loop(0, (NBN + NS - 1) // NS, body, 0)


def _sc_gine_aggr(tab4, srcs4, dst, epq, zer, Dq):
    per = E // NS
    CH = 160
    SUB = 80
    nch = per // CH

    @functools.partial(
        pl.kernel, mesh=_MESH,
        compiler_params=pltpu.CompilerParams(use_tc_tiling_on_sc=False),
        out_type=jax.ShapeDtypeStruct((NQ * N, Dq), F32),
        scratch_types=[
            pltpu.VMEM((SUB,), jnp.int32), pltpu.VMEM((SUB,), jnp.int32),
            pltpu.VMEM((SUB,), jnp.int32), pltpu.VMEM((SUB,), jnp.int32),
            pltpu.VMEM((CH, Dq), F32), pltpu.VMEM((CH, Dq), F32),
            pltpu.VMEM_SHARED((N, Dq), F32),
            pltpu.SemaphoreType.DMA, pltpu.SemaphoreType.DMA,
        ],
    )
    def k(tab_r, src_r, dst_r, ep0_r, ep1_r, ep2_r, ep3_r, zer_r, out_r,
          is0, is1, id0, id1, hr, epb, acc, sem0, sem1):
        c = lax.axis_index("c")
        s = lax.axis_index("s")
        base = s * per

        def round_(ep_r, q):
            _acc_rows(s, lambda r0: pltpu.sync_copy(zer_r.at[pl.ds(r0, ZB)],
                                                    acc.at[pl.ds(r0, ZB)]))
            plsc.subcore_barrier()

            def chunk(i, _):
                e0 = base + i * CH
                pltpu.sync_copy(src_r.at[pl.ds(q * E + e0, SUB)], is0)
                pltpu.sync_copy(src_r.at[pl.ds(q * E + e0 + SUB, SUB)], is1)
                pltpu.sync_copy(dst_r.at[pl.ds(e0, SUB)], id0)
                pltpu.sync_copy(dst_r.at[pl.ds(e0 + SUB, SUB)], id1)
                g0 = pltpu.async_copy(tab_r.at[is0], hr.at[pl.ds(0, SUB)],
                                      sem0)
                g1 = pltpu.async_copy(tab_r.at[is1], hr.at[pl.ds(SUB, SUB)],
                                      sem1)
                pltpu.sync_copy(ep_r.at[pl.ds(e0, CH)], epb)
                g0.wait()
                g1.wait()

                def row(r, _):
                    for j in range(Dq // 16):
                        sl = pl.ds(j * 16, 16)
                        epb[r, sl] = jnp.maximum(hr[r, sl] + epb[r, sl], 0.0)
                    return 0

                lax.fori_loop(0, CH, row, 0)
                pltpu.sync_copy(epb.at[pl.ds(0, SUB)], acc.at[id0], add=True)
                pltpu.sync_copy(epb.at[pl.ds(SUB, SUB)], acc.at[id1],
                                add=True)
                return 0

            lax.fori_loop(0, nch, chunk, 0)
            plsc.subcore_barrier()
            _acc_rows(s, lambda r0: pltpu.sync_copy(
                acc.at[pl.ds(r0, ZB)], out_r.at[pl.ds(q * N + r0, ZB)]))
            plsc.subcore_barrier()

        @pl.when(c == 0)
        def _():
            round_(ep0_r, 0)
            round_(ep1_r, 1)

        @pl.when(c == 1)
        def _():
            round_(ep2_r, 2)
            round_(ep3_r, 3)

    return k(tab4, srcs4, dst, *epq, zer)


def _sc_gather2(tabA, tabB, idxA, idxB):
    per = E // NW
    CH = 80
    nch = per // CH

    @functools.partial(
        pl.kernel, mesh=_MESH,
        compiler_params=pltpu.CompilerParams(use_tc_tiling_on_sc=False),
        out_type=[jax.ShapeDtypeStruct((E, HID), F32),
                  jax.ShapeDtypeStruct((E, HID), F32)],
        scratch_types=[
            pltpu.VMEM((CH,), jnp.int32), pltpu.VMEM((CH,), jnp.int32),
            pltpu.VMEM((CH, HID), F32), pltpu.VMEM((CH, HID), F32),
            pltpu.SemaphoreType.DMA, pltpu.SemaphoreType.DMA,
        ],
    )
    def k(tA, tB, iA, iB, oA, oB, iva, ivb, bA, bB, s0, s1):
        c = lax.axis_index("c")
        s = lax.axis_index("s")
        wid = s * NC + c
        base = wid * per

        def chunk(i, _):
            e0 = base + i * CH
            pltpu.sync_copy(iA.at[pl.ds(e0, CH)], iva)
            pltpu.sync_copy(iB.at[pl.ds(e0, CH)], ivb)
            ga = pltpu.async_copy(tA.at[iva], bA, s0)
            gb = pltpu.async_copy(tB.at[ivb], bB, s1)
            ga.wait()
            pltpu.sync_copy(bA, oA.at[pl.ds(e0, CH)])
            gb.wait()
            pltpu.sync_copy(bB, oB.at[pl.ds(e0, CH)])
            return 0

        lax.fori_loop(0, nch, chunk, 0)

    return k(tabA, tabB, idxA, idxB)


def _sc_scatter_q(rowsq, dst, zer, Dq):
    per = E // NS
    CH = 160
    SUB = 80
    nch = per // CH

    @functools.partial(
        pl.kernel, mesh=_MESH,
        compiler_params=pltpu.CompilerParams(use_tc_tiling_on_sc=False),
        out_type=jax.ShapeDtypeStruct((NQ * N, Dq), F32),
        scratch_types=[
            pltpu.VMEM((SUB,), jnp.int32), pltpu.VMEM((SUB,), jnp.int32),
            pltpu.VMEM((CH, Dq), F32),
            pltpu.VMEM_SHARED((N, Dq), F32),
        ],
    )
    def k(r0_r, r1_r, r2_r, r3_r, dst_r, zer_r, out_r, id0, id1, buf, acc):
        c = lax.axis_index("c")
        s = lax.axis_index("s")
        base = s * per

        def round_(rows_r, q):
            _acc_rows(s, lambda r0: pltpu.sync_copy(zer_r.at[pl.ds(r0, ZB)],
                                                    acc.at[pl.ds(r0, ZB)]))
            plsc.subcore_barrier()

            def chunk(i, _):
                e0 = base + i * CH
                pltpu.sync_copy(dst_r.at[pl.ds(e0, SUB)], id0)
                pltpu.sync_copy(dst_r.at[pl.ds(e0 + SUB, SUB)], id1)
                pltpu.sync_copy(rows_r.at[pl.ds(e0, CH)], buf)
                pltpu.sync_copy(buf.at[pl.ds(0, SUB)], acc.at[id0], add=True)
                pltpu.sync_copy(buf.at[pl.ds(SUB, SUB)], acc.at[id1],
                                add=True)
                return 0

            lax.fori_loop(0, nch, chunk, 0)
            plsc.subcore_barrier()
            _acc_rows(s, lambda r0: pltpu.sync_copy(
                acc.at[pl.ds(r0, ZB)], out_r.at[pl.ds(q * N + r0, ZB)]))
            plsc.subcore_barrier()

        @pl.when(c == 0)
        def _():
            round_(r0_r, 0)
            round_(r1_r, 1)

        @pl.when(c == 1)
        def _():
            round_(r2_r, 2)
            round_(r3_r, 3)

    return k(*rowsq, dst, zer)


def _sc_scatter_den(ex16, dst, zer16):
    per = E // NW
    CH = 160
    SUB = 80
    nch = per // CH

    @functools.partial(
        pl.kernel, mesh=_MESH,
        compiler_params=pltpu.CompilerParams(use_tc_tiling_on_sc=False),
        out_type=jax.ShapeDtypeStruct((2 * N, NHP), F32),
        scratch_types=[
            pltpu.VMEM((SUB,), jnp.int32), pltpu.VMEM((SUB,), jnp.int32),
            pltpu.VMEM((CH, NHP), F32),
            pltpu.VMEM_SHARED((N, NHP), F32),
        ],
    )
    def k(ex_r, dst_r, zer_r, out_r, id0, id1, buf, acc):
        c = lax.axis_index("c")
        s = lax.axis_index("s")
        _acc_rows(s, lambda r0: pltpu.sync_copy(zer_r.at[pl.ds(r0, ZB)],
                                                acc.at[pl.ds(r0, ZB)]))
        plsc.subcore_barrier()
        base = (c * NS + s) * per

        def chunk(i, _):
            e0 = base + i * CH
            pltpu.sync_copy(dst_r.at[pl.ds(e0, SUB)], id0)
            pltpu.sync_copy(dst_r.at[pl.ds(e0 + SUB, SUB)], id1)
            pltpu.sync_copy(ex_r.at[pl.ds(e0, CH)], buf)
            pltpu.sync_copy(buf.at[pl.ds(0, SUB)], acc.at[id0], add=True)
            pltpu.sync_copy(buf.at[pl.ds(SUB, SUB)], acc.at[id1], add=True)
            return 0

        lax.fori_loop(0, nch, chunk, 0)
        plsc.subcore_barrier()
        _acc_rows(s, lambda r0: pltpu.sync_copy(
            acc.at[pl.ds(r0, ZB)], out_r.at[pl.ds(c * N + r0, ZB)]))

    return k(ex16, dst, zer16)


def _sc_gather_center(tab, idx):
    per = 256 // NW

    @functools.partial(
        pl.kernel, mesh=_MESH,
        compiler_params=pltpu.CompilerParams(use_tc_tiling_on_sc=False),
        out_type=jax.ShapeDtypeStruct((256, HID), F32),
        scratch_types=[
            pltpu.VMEM((per,), jnp.int32),
            pltpu.VMEM((per, HID), F32),
            pltpu.SemaphoreType.DMA,
        ],
    )
    def k(t_r, i_r, o_r, iv, buf, sem):
        c = lax.axis_index("c")
        s = lax.axis_index("s")
        wid = s * NC + c
        base = wid * per
        pltpu.sync_copy(i_r.at[pl.ds(base, per)], iv)
        pltpu.async_copy(t_r.at[iv], buf, sem).wait()
        pltpu.sync_copy(buf, o_r.at[pl.ds(base, per)])

    return k(tab, idx)


def _stack_quarters(h):
    D = h.shape[1]
    Dq = D // NQ
    return jnp.concatenate([h[:, q * Dq:(q + 1) * Dq] for q in range(NQ)],
                           axis=0)


def kernel(x, edge_attr, edge_index, center_id, ee_W1, ee_b1, ee_W2, ee_b2,
           g0_eW, g0_eb, g0_W1, g0_b1, g0_W2, g0_b2,
           g1_eW, g1_eb, g1_W1, g1_b1, g1_W2, g1_b2,
           g2_eW, g2_eb, g2_W1, g2_b1, g2_W2, g2_b2,
           gat_Wl, gat_bl, gat_Wr, gat_br, gat_We, gat_att, gat_b,
           mu_W, mu_b, lv_W, lv_b,
           ln0_g, ln0_b, ln1_g, ln1_b, ln2_g, ln2_b, ln3_g, ln3_b):
    src = edge_index[0]
    dst = edge_index[1]
    srcs4 = jnp.concatenate([src + q * N for q in range(NQ)])
    r2 = lambda a: a[None, :]

    Wcat = jnp.concatenate([g0_eW, g1_eW, g2_eW, gat_We], axis=1)
    bcat = jnp.concatenate([g0_eb, g1_eb, g2_eb,
                            jnp.zeros((HID,), F32)])[None, :]
    hh = jnp.repeat(jnp.arange(NH), HC)
    A = jnp.zeros((HID, NHP), F32).at[jnp.arange(HID), hh].set(
        gat_att.reshape(-1))
    R = jnp.zeros((NHP, HID), F32).at[hh, jnp.arange(HID)].set(1.0)

    zer_q = jnp.zeros((N, QH), F32)
    zer_0 = jnp.zeros((N, Q0), F32)
    zer16 = jnp.zeros((N, NHP), F32)

    eps = _edge_encode_project(
        edge_attr, ee_W1, r2(ee_b1), ee_W2, r2(ee_b2), Wcat, bcat)
    ep0q = eps[0:NQ]
    ep1q = eps[NQ:2 * NQ]
    ep2q = eps[2 * NQ:3 * NQ]
    epgq = eps[3 * NQ:4 * NQ]

    agg0 = _sc_gine_aggr(_stack_quarters(x), srcs4, dst, ep0q, zer_0, Q0)
    h = _gine_node(x, agg0, None, g0_W1, r2(g0_b1), g0_W2, r2(g0_b2),
                   r2(ln0_g), r2(ln0_b), has_res=False)

    agg1 = _sc_gine_aggr(_stack_quarters(h), srcs4, dst, ep1q, zer_q, QH)
    h = _gine_node(h, agg1, h, g1_W1, r2(g1_b1), g1_W2, r2(g1_b2),
                   r2(ln1_g), r2(ln1_b), has_res=True)
    agg2 = _sc_gine_aggr(_stack_quarters(h), srcs4, dst, ep2q, zer_q, QH)
    h = _gine_node(h, agg2, h, g2_W1, r2(g2_b1), g2_W2, r2(g2_b2),
                   r2(ln2_g), r2(ln2_b), has_res=True)

    xl, xr = _gat_proj(h, gat_Wl, r2(gat_bl), gat_Wr, r2(gat_br))
    xlg, xrg = _sc_gather2(xl, xr, src, dst)
    outs = _gat_edge(xlg, xrg, epgq, A, R)
    ex16, wmsgq = outs[0], outs[1:]
    den2 = _sc_scatter_den(ex16, dst, zer16)
    numer4 = _sc_scatter_q(wmsgq, dst, zer_q, QH)
    aggrg4 = _sc_scatter_q(epgq, dst, zer_q, QH)
    h = _gat_node(h, xl, xr, aggrg4, numer4, den2, A, R, r2(gat_b),
                  r2(ln3_g), r2(ln3_b))

    cv = _sc_gather_center(h, center_id)
    mu, logvar = _heads(cv, mu_W, r2(mu_b), lv_W, r2(lv_b))
    return mu, logvar


# speedup vs baseline: 5.4101x; 1.0857x over previous
import functools

import jax
import jax.numpy as jnp
from jax import lax
from jax.experimental import pallas as pl
from jax.experimental.pallas import tpu as pltpu
from jax.experimental.pallas import tpu_sc as plsc

N = 10000
E = 320000
F_IN = 128
F_E = 16
HID = 384
LAT = 64
NH = 8
HC = 48
NHP = 16
F32 = jnp.float32

NC = 2
NS = 16
NW = NC * NS
PHI = jax.lax.Precision.HIGHEST
NQ = 4
QH = HID // NQ
Q0 = F_IN // NQ

ZB = 80
NBN = N // ZB


def _edge_encode_project(ea, W1, b1, W2, b2, Wcat, bcat):
    Be = 512
    nb = E // Be

    def body(ea_ref, w1, bb1, w2, bb2, wc, bc, *outs):
        u = jnp.maximum(jnp.dot(ea_ref[...], w1[...],
                                preferred_element_type=F32) + bb1[...], 0.0)
        e = jnp.dot(u, w2[...], preferred_element_type=F32) + bb2[...]
        proj = jnp.dot(e, wc[...], preferred_element_type=F32) + bc[...]
        for q in range(NQ):
            outs[q][...] = proj[:, q * Q0:(q + 1) * Q0]
        for k in range(3 * NQ):
            outs[NQ + k][...] = proj[:, F_IN + k * QH:F_IN + (k + 1) * QH]

    full = lambda r, c: pl.BlockSpec((r, c), lambda i: (0, 0))
    eb = lambda c: pl.BlockSpec((Be, c), lambda i: (i, 0))
    sh = lambda c: jax.ShapeDtypeStruct((E, c), F32)
    return pl.pallas_call(
        body,
        grid=(nb,),
        in_specs=[
            pl.BlockSpec((Be, F_E), lambda i: (i, 0)),
            full(F_E, HID), full(1, HID), full(HID, HID), full(1, HID),
            full(HID, F_IN + 3 * HID), full(1, F_IN + 3 * HID),
        ],
        out_specs=[eb(Q0)] * NQ + [eb(QH)] * (3 * NQ),
        out_shape=[sh(Q0)] * NQ + [sh(QH)] * (3 * NQ),
    )(ea, W1, b1, W2, b2, Wcat, bcat)


def _gine_node(hin, agg4, res, W1, b1, W2, b2, g, b, has_res):
    Din = hin.shape[1]
    Dq = Din // NQ
    Bn = 1000
    nb = N // Bn
    nblk = N // Bn

    def body(*refs):
        if has_res:
            (hin_r, a0, a1, a2, a3, res_r, w1, bb1, w2, bb2, gg, bb,
             out) = refs
        else:
            (hin_r, a0, a1, a2, a3, w1, bb1, w2, bb2, gg, bb, out) = refs
        aggr = jnp.concatenate([a0[...], a1[...], a2[...], a3[...]], axis=1)
        z = hin_r[...] + aggr
        u = jnp.maximum(jnp.dot(z, w1[...], preferred_element_type=F32)
                        + bb1[...], 0.0)
        t = jnp.dot(u, w2[...], preferred_element_type=F32) + bb2[...]
        if has_res:
            t = t + res_r[...]
        m = jnp.mean(t, axis=1, keepdims=True)
        d = t - m
        v = jnp.mean(d * d, axis=1, keepdims=True)
        out[...] = d / jnp.sqrt(v + 1e-5) * gg[...] + bb[...]

    full = lambda r, c: pl.BlockSpec((r, c), lambda i: (0, 0))
    aview = lambda q: pl.BlockSpec(
        (Bn, Dq), lambda i, q=q: (i + q * nblk, 0))
    in_specs = [pl.BlockSpec((Bn, Din), lambda i: (i, 0)),
                aview(0), aview(1), aview(2), aview(3)]
    args = [hin, agg4, agg4, agg4, agg4]
    if has_res:
        in_specs.append(pl.BlockSpec((Bn, HID), lambda i: (i, 0)))
        args.append(res)
    in_specs += [full(Din, HID), full(1, HID), full(HID, HID), full(1, HID),
                 full(1, HID), full(1, HID)]
    args += [W1, b1, W2, b2, g, b]
    return pl.pallas_call(
        body,
        grid=(nb,),
        in_specs=in_specs,
        out_specs=pl.BlockSpec((Bn, HID), lambda i: (i, 0)),
        out_shape=jax.ShapeDtypeStruct((N, HID), F32),
    )(*args)


def _gat_proj(h, Wl, bl, Wr, br):
    Bn = 1000
    nb = N // Bn

    def body(h_r, wl, bbl, wr, bbr, oxl, oxr):
        hv = h_r[...]
        oxl[...] = jnp.dot(hv, wl[...], preferred_element_type=F32) + bbl[...]
        oxr[...] = jnp.dot(hv, wr[...], preferred_element_type=F32) + bbr[...]

    full = lambda r, c: pl.BlockSpec((r, c), lambda i: (0, 0))
    return pl.pallas_call(
        body,
        grid=(nb,),
        in_specs=[pl.BlockSpec((Bn, HID), lambda i: (i, 0)),
                  full(HID, HID), full(1, HID), full(HID, HID), full(1, HID)],
        out_specs=[pl.BlockSpec((Bn, HID), lambda i: (i, 0)),
                   pl.BlockSpec((Bn, HID), lambda i: (i, 0))],
        out_shape=[jax.ShapeDtypeStruct((N, HID), F32),
                   jax.ShapeDtypeStruct((N, HID), F32)],
    )(h, Wl, bl, Wr, br)


def _gat_edge(xlg, xrg, epgq, A, R):
    Be = 512
    nb = E // Be

    def body(xl_r, xr_r, e0, e1, e2, e3, a_r, r_r, oex, *ows):
        xlv = xl_r[...]
        epv = jnp.concatenate([e0[...], e1[...], e2[...], e3[...]], axis=1)
        q = xlv + xr_r[...] + epv
        ql = jnp.where(q >= 0.0, q, 0.2 * q)
        al = jnp.dot(ql, a_r[...], preferred_element_type=F32, precision=PHI)
        ex = jnp.exp(al)
        oex[...] = ex
        w = xlv * jnp.dot(ex, r_r[...], preferred_element_type=F32, precision=PHI)
        for k in range(NQ):
            ows[k][...] = w[:, k * QH:(k + 1) * QH]

    full = lambda r, c: pl.BlockSpec((r, c), lambda i: (0, 0))
    eb = lambda c: pl.BlockSpec((Be, c), lambda i: (i, 0))
    return pl.pallas_call(
        body,
        grid=(nb,),
        in_specs=[eb(HID), eb(HID)] + [eb(QH)] * NQ
        + [full(HID, NHP), full(NHP, HID)],
        out_specs=[eb(NHP)] + [eb(QH)] * NQ,
        out_shape=[jax.ShapeDtypeStruct((E, NHP), F32)]
        + [jax.ShapeDtypeStruct((E, QH), F32)] * NQ,
    )(xlg, xrg, *epgq, A, R)


def _gat_node(h, xl, xr, aggrg4, numer4, den2, A, R, gat_b, g, b):
    Bn = 1000
    nb = N // Bn
    nblk = N // Bn

    def body(h_r, xl_r, xr_r, ag0, ag1, ag2, ag3, nm0, nm1, nm2, nm3,
             d0, d1, a_r, r_r, gb, gg, bb, out):
        xlv = xl_r[...]
        den_sc = d0[...] + d1[...]
        deg = jnp.maximum(den_sc[:, 8:9], 1.0)
        lep = jnp.concatenate([ag0[...], ag1[...], ag2[...], ag3[...]],
                              axis=1) / deg
        q = xlv + xr_r[...] + lep
        ql = jnp.where(q >= 0.0, q, 0.2 * q)
        exl = jnp.exp(jnp.dot(ql, a_r[...], preferred_element_type=F32, precision=PHI))
        den_t = den_sc + exl + 1e-16
        numer = jnp.concatenate([nm0[...], nm1[...], nm2[...], nm3[...]],
                                axis=1) \
            + xlv * jnp.dot(exl, r_r[...], preferred_element_type=F32, precision=PHI)
        den_rep = jnp.dot(den_t, r_r[...], preferred_element_type=F32, precision=PHI)
        gat = numer / den_rep + gb[...]
        t = h_r[...] + gat
        m = jnp.mean(t, axis=1, keepdims=True)
        d = t - m
        v = jnp.mean(d * d, axis=1, keepdims=True)
        out[...] = d / jnp.sqrt(v + 1e-5) * gg[...] + bb[...]

    full = lambda r, c: pl.BlockSpec((r, c), lambda i: (0, 0))
    qv = lambda q: pl.BlockSpec((Bn, QH), lambda i, q=q: (i + q * nblk, 0))
    hv = lambda: pl.BlockSpec((Bn, HID), lambda i: (i, 0))
    return pl.pallas_call(
        body,
        grid=(nb,),
        in_specs=[hv(), hv(), hv(), qv(0), qv(1), qv(2), qv(3),
                  qv(0), qv(1), qv(2), qv(3),
                  pl.BlockSpec((Bn, NHP), lambda i: (i, 0)),
                  pl.BlockSpec((Bn, NHP), lambda i: (i + nblk, 0)),
                  full(HID, NHP), full(NHP, HID), full(1, HID),
                  full(1, HID), full(1, HID)],
        out_specs=pl.BlockSpec((Bn, HID), lambda i: (i, 0)),
        out_shape=jax.ShapeDtypeStruct((N, HID), F32),
    )(h, xl, xr, aggrg4, aggrg4, aggrg4, aggrg4,
      numer4, numer4, numer4, numer4, den2, den2,
      A, R, gat_b, g, b)


def _heads(cv, mu_W, mu_b, lv_W, lv_b):
    def body(cv_r, wm, bm, wv, bv, omu, olv):
        c = cv_r[...]
        omu[...] = jnp.dot(c, wm[...], preferred_element_type=F32) + bm[...]
        olv[...] = jnp.dot(c, wv[...], preferred_element_type=F32) + bv[...]

    full = lambda r, c: pl.BlockSpec((r, c), lambda i: (0, 0))
    return pl.pallas_call(
        body,
        grid=(1,),
        in_specs=[pl.BlockSpec((256, HID), lambda i: (0, 0)),
                  full(HID, LAT), full(1, LAT), full(HID, LAT), full(1, LAT)],
        out_specs=[pl.BlockSpec((256, LAT), lambda i: (0, 0)),
                   pl.BlockSpec((256, LAT), lambda i: (0, 0))],
        out_shape=[jax.ShapeDtypeStruct((256, LAT), F32),
                   jax.ShapeDtypeStruct((256, LAT), F32)],
    )(cv, mu_W, mu_b, lv_W, lv_b)


_MESH = plsc.VectorSubcoreMesh(core_axis_name="c", subcore_axis_name="s")


def _acc_rows(s, fn):
    def body(k, _):
        blk = s + k * NS

        @pl.when(blk < NBN)
        def _():
            fn(blk * ZB)
        return 0

    lax.fori_loop(0, (NBN + NS - 1) // NS, body, 0)


def _sc_gine_aggr(tab4, srcs4, dst, epq, zer, Dq):
    per = E // NS
    CH = 160
    SUB = 80
    nch = per // CH
    assert nch * CH == per

    @functools.partial(
        pl.kernel, mesh=_MESH,
        compiler_params=pltpu.CompilerParams(use_tc_tiling_on_sc=False),
        out_type=jax.ShapeDtypeStruct((NQ * N, Dq), F32),
        scratch_types=[
            [pltpu.VMEM((SUB,), jnp.int32)] * 4,
            [pltpu.VMEM((SUB,), jnp.int32)] * 4,
            [pltpu.VMEM((CH, Dq), F32)] * 2,
            [pltpu.VMEM((CH, Dq), F32)] * 2,
            pltpu.VMEM_SHARED((N, Dq), F32),
            [pltpu.SemaphoreType.DMA] * 2,
            [pltpu.SemaphoreType.DMA] * 2,
            [pltpu.SemaphoreType.DMA] * 2,
        ],
    )
    def k(tab_r, src_r, dst_r, ep0_r, ep1_r, ep2_r, ep3_r, zer_r, out_r,
          iss, ids, hrs, epbs, acc, semg0, semg1, seme):
        c = lax.axis_index("c")
        s = lax.axis_index("s")
        base = s * per

        def round_(ep_r, q):
            _acc_rows(s, lambda r0: pltpu.sync_copy(zer_r.at[pl.ds(r0, ZB)],
                                                    acc.at[pl.ds(r0, ZB)]))
            plsc.subcore_barrier()

            def stage_a(i, b):
                e0 = base + i * CH
                pltpu.sync_copy(src_r.at[pl.ds(q * E + e0, SUB)], iss[2 * b])
                pltpu.sync_copy(src_r.at[pl.ds(q * E + e0 + SUB, SUB)],
                                iss[2 * b + 1])
                pltpu.sync_copy(dst_r.at[pl.ds(e0, SUB)], ids[2 * b])
                pltpu.sync_copy(dst_r.at[pl.ds(e0 + SUB, SUB)],
                                ids[2 * b + 1])
                pltpu.async_copy(tab_r.at[iss[2 * b]],
                                 hrs[b].at[pl.ds(0, SUB)], semg0[b])
                pltpu.async_copy(tab_r.at[iss[2 * b + 1]],
                                 hrs[b].at[pl.ds(SUB, SUB)], semg1[b])
                pltpu.async_copy(ep_r.at[pl.ds(e0, CH)], epbs[b], seme[b])

            def stage_b(b):
                hr, epb = hrs[b], epbs[b]
                pltpu.make_async_copy(tab_r.at[iss[2 * b]],
                                      hr.at[pl.ds(0, SUB)], semg0[b]).wait()
                pltpu.make_async_copy(tab_r.at[iss[2 * b + 1]],
                                      hr.at[pl.ds(SUB, SUB)],
                                      semg1[b]).wait()
                pltpu.make_async_copy(ep_r.at[pl.ds(0, CH)], epb,
                                      seme[b]).wait()

                def row(r, _):
                    for j in range(Dq // 16):
                        sl = pl.ds(j * 16, 16)
                        epb[r, sl] = jnp.maximum(hr[r, sl] + epb[r, sl], 0.0)
                    return 0

                lax.fori_loop(0, CH, row, 0)
                pltpu.sync_copy(epb.at[pl.ds(0, SUB)], acc.at[ids[2 * b]],
                                add=True)
                pltpu.sync_copy(epb.at[pl.ds(SUB, SUB)],
                                acc.at[ids[2 * b + 1]], add=True)

            stage_a(0, 0)

            def pair(kk, _):
                for sub in (0, 1):
                    i = 2 * kk + sub

                    @pl.when(i + 1 < nch)
                    def _():
                        stage_a(i + 1, 1 - sub)

                    @pl.when(i < nch)
                    def _():
                        stage_b(sub)
                return 0

            lax.fori_loop(0, (nch + 1) // 2, pair, 0)
            plsc.subcore_barrier()
            _acc_rows(s, lambda r0: pltpu.sync_copy(
                acc.at[pl.ds(r0, ZB)], out_r.at[pl.ds(q * N + r0, ZB)]))
            plsc.subcore_barrier()

        @pl.when(c == 0)
        def _():
            round_(ep0_r, 0)
            round_(ep1_r, 1)

        @pl.when(c == 1)
        def _():
            round_(ep2_r, 2)
            round_(ep3_r, 3)

    return k(tab4, srcs4, dst, *epq, zer)


def _sc_gather2(tabA, tabB, idxA, idxB):
    per = E // NW
    CH = 80
    nch = per // CH
    assert nch * CH == per

    @functools.partial(
        pl.kernel, mesh=_MESH,
        compiler_params=pltpu.CompilerParams(use_tc_tiling_on_sc=False),
        out_type=[jax.ShapeDtypeStruct((E, HID), F32),
                  jax.ShapeDtypeStruct((E, HID), F32)],
        scratch_types=[
            [pltpu.VMEM((CH,), jnp.int32)] * 2,
            [pltpu.VMEM((CH,), jnp.int32)] * 2,
            [pltpu.VMEM((CH, HID), F32)] * 2,
            [pltpu.VMEM((CH, HID), F32)] * 2,
            [pltpu.SemaphoreType.DMA] * 2,
            [pltpu.SemaphoreType.DMA] * 2,
        ],
    )
    def k(tA, tB, iA, iB, oA, oB, ivas, ivbs, bAs, bBs, s0s, s1s):
        c = lax.axis_index("c")
        s = lax.axis_index("s")
        wid = s * NC + c
        base = wid * per

        def stage_a(i, b):
            e0 = base + i * CH
            pltpu.sync_copy(iA.at[pl.ds(e0, CH)], ivas[b])
            pltpu.sync_copy(iB.at[pl.ds(e0, CH)], ivbs[b])
            pltpu.async_copy(tA.at[ivas[b]], bAs[b], s0s[b])
            pltpu.async_copy(tB.at[ivbs[b]], bBs[b], s1s[b])

        def stage_b(i, b):
            e0 = base + i * CH
            pltpu.make_async_copy(tA.at[ivas[b]], bAs[b], s0s[b]).wait()
            pltpu.sync_copy(bAs[b], oA.at[pl.ds(e0, CH)])
            pltpu.make_async_copy(tB.at[ivbs[b]], bBs[b], s1s[b]).wait()
            pltpu.sync_copy(bBs[b], oB.at[pl.ds(e0, CH)])

        stage_a(0, 0)

        def pair(kk, _):
            for sub in (0, 1):
                i = 2 * kk + sub

                @pl.when(i + 1 < nch)
                def _():
                    stage_a(i + 1, 1 - sub)

                @pl.when(i < nch)
                def _():
                    stage_b(i, sub)
            return 0

        lax.fori_loop(0, (nch + 1) // 2, pair, 0)

    return k(tabA, tabB, idxA, idxB)


def _sc_scatter_q(rowsq, dst, zer, Dq):
    per = E // NS
    CH = 160
    SUB = 80
    nch = per // CH

    @functools.partial(
        pl.kernel, mesh=_MESH,
        compiler_params=pltpu.CompilerParams(use_tc_tiling_on_sc=False),
        out_type=jax.ShapeDtypeStruct((NQ * N, Dq), F32),
        scratch_types=[
            [pltpu.VMEM((SUB,), jnp.int32)] * 4,
            [pltpu.VMEM((CH, Dq), F32)] * 2,
            pltpu.VMEM_SHARED((N, Dq), F32),
            [pltpu.SemaphoreType.DMA] * 2,
        ],
    )
    def k(r0_r, r1_r, r2_r, r3_r, dst_r, zer_r, out_r, ids, bufs, acc, sems):
        c = lax.axis_index("c")
        s = lax.axis_index("s")
        base = s * per

        def round_(rows_r, q):
            _acc_rows(s, lambda r0: pltpu.sync_copy(zer_r.at[pl.ds(r0, ZB)],
                                                    acc.at[pl.ds(r0, ZB)]))
            plsc.subcore_barrier()

            def stage_a(i, b):
                e0 = base + i * CH
                pltpu.sync_copy(dst_r.at[pl.ds(e0, SUB)], ids[2 * b])
                pltpu.sync_copy(dst_r.at[pl.ds(e0 + SUB, SUB)],
                                ids[2 * b + 1])
                pltpu.async_copy(rows_r.at[pl.ds(e0, CH)], bufs[b], sems[b])

            def stage_b(b):
                pltpu.make_async_copy(rows_r.at[pl.ds(0, CH)], bufs[b],
                                      sems[b]).wait()
                pltpu.sync_copy(bufs[b].at[pl.ds(0, SUB)],
                                acc.at[ids[2 * b]], add=True)
                pltpu.sync_copy(bufs[b].at[pl.ds(SUB, SUB)],
                                acc.at[ids[2 * b + 1]], add=True)

            stage_a(0, 0)

            def pair(kk, _):
                for sub in (0, 1):
                    i = 2 * kk + sub

                    @pl.when(i + 1 < nch)
                    def _():
                        stage_a(i + 1, 1 - sub)

                    @pl.when(i < nch)
                    def _():
                        stage_b(sub)
                return 0

            lax.fori_loop(0, (nch + 1) // 2, pair, 0)
            plsc.subcore_barrier()
            _acc_rows(s, lambda r0: pltpu.sync_copy(
                acc.at[pl.ds(r0, ZB)], out_r.at[pl.ds(q * N + r0, ZB)]))
            plsc.subcore_barrier()

        @pl.when(c == 0)
        def _():
            round_(r0_r, 0)
            round_(r1_r, 1)

        @pl.when(c == 1)
        def _():
            round_(r2_r, 2)
            round_(r3_r, 3)

    return k(*rowsq, dst, zer)


def _sc_scatter_den(ex16, dst, zer16):
    per = E // NW
    CH = 80
    SUB = 40
    nch = per // CH
    assert nch * CH == per

    @functools.partial(
        pl.kernel, mesh=_MESH,
        compiler_params=pltpu.CompilerParams(use_tc_tiling_on_sc=False),
        out_type=jax.ShapeDtypeStruct((2 * N, NHP), F32),
        scratch_types=[
            [pltpu.VMEM((SUB,), jnp.int32)] * 4,
            [pltpu.VMEM((CH, NHP), F32)] * 2,
            pltpu.VMEM_SHARED((N, NHP), F32),
            [pltpu.SemaphoreType.DMA] * 2,
        ],
    )
    def k(ex_r, dst_r, zer_r, out_r, ids, bufs, acc, sems):
        c = lax.axis_index("c")
        s = lax.axis_index("s")
        _acc_rows(s, lambda r0: pltpu.sync_copy(zer_r.at[pl.ds(r0, ZB)],
                                                acc.at[pl.ds(r0, ZB)]))
        plsc.subcore_barrier()
        base = (c * NS + s) * per

        def stage_a(i, b):
            e0 = base + i * CH
            pltpu.sync_copy(dst_r.at[pl.ds(e0, SUB)], ids[2 * b])
            pltpu.sync_copy(dst_r.at[pl.ds(e0 + SUB, SUB)], ids[2 * b + 1])
            pltpu.async_copy(ex_r.at[pl.ds(e0, CH)], bufs[b], sems[b])

        def stage_b(b):
            pltpu.make_async_copy(ex_r.at[pl.ds(0, CH)], bufs[b],
                                  sems[b]).wait()
            pltpu.sync_copy(bufs[b].at[pl.ds(0, SUB)], acc.at[ids[2 * b]],
                            add=True)
            pltpu.sync_copy(bufs[b].at[pl.ds(SUB, SUB)],
                            acc.at[ids[2 * b + 1]], add=True)

        stage_a(0, 0)

        def pair(kk, _):
            for sub in (0, 1):
                i = 2 * kk + sub

                @pl.when(i + 1 < nch)
                def _():
                    stage_a(i + 1, 1 - sub)

                @pl.when(i < nch)
                def _():
                    stage_b(sub)
            return 0

        lax.fori_loop(0, (nch + 1) // 2, pair, 0)
        plsc.subcore_barrier()
        _acc_rows(s, lambda r0: pltpu.sync_copy(
            acc.at[pl.ds(r0, ZB)], out_r.at[pl.ds(c * N + r0, ZB)]))

    return k(ex16, dst, zer16)


def _sc_gather_center(tab, idx):
    per = 256 // NW

    @functools.partial(
        pl.kernel, mesh=_MESH,
        compiler_params=pltpu.CompilerParams(use_tc_tiling_on_sc=False),
        out_type=jax.ShapeDtypeStruct((256, HID), F32),
        scratch_types=[
            pltpu.VMEM((per,), jnp.int32),
            pltpu.VMEM((per, HID), F32),
            pltpu.SemaphoreType.DMA,
        ],
    )
    def k(t_r, i_r, o_r, iv, buf, sem):
        c = lax.axis_index("c")
        s = lax.axis_index("s")
        wid = s * NC + c
        base = wid * per
        pltpu.sync_copy(i_r.at[pl.ds(base, per)], iv)
        pltpu.async_copy(t_r.at[iv], buf, sem).wait()
        pltpu.sync_copy(buf, o_r.at[pl.ds(base, per)])

    return k(tab, idx)


def _stack_quarters(h):
    D = h.shape[1]
    Dq = D // NQ
    return jnp.concatenate([h[:, q * Dq:(q + 1) * Dq] for q in range(NQ)],
                           axis=0)


def kernel(x, edge_attr, edge_index, center_id, ee_W1, ee_b1, ee_W2, ee_b2,
           g0_eW, g0_eb, g0_W1, g0_b1, g0_W2, g0_b2,
           g1_eW, g1_eb, g1_W1, g1_b1, g1_W2, g1_b2,
           g2_eW, g2_eb, g2_W1, g2_b1, g2_W2, g2_b2,
           gat_Wl, gat_bl, gat_Wr, gat_br, gat_We, gat_att, gat_b,
           mu_W, mu_b, lv_W, lv_b,
           ln0_g, ln0_b, ln1_g, ln1_b, ln2_g, ln2_b, ln3_g, ln3_b):
    src = edge_index[0]
    dst = edge_index[1]
    srcs4 = jnp.concatenate([src + q * N for q in range(NQ)])
    r2 = lambda a: a[None, :]

    Wcat = jnp.concatenate([g0_eW, g1_eW, g2_eW, gat_We], axis=1)
    bcat = jnp.concatenate([g0_eb, g1_eb, g2_eb,
                            jnp.zeros((HID,), F32)])[None, :]
    hh = jnp.repeat(jnp.arange(NH), HC)
    A = jnp.zeros((HID, NHP), F32).at[jnp.arange(HID), hh].set(
        gat_att.reshape(-1))
    R = jnp.zeros((NHP, HID), F32).at[hh, jnp.arange(HID)].set(1.0)

    zer_q = jnp.zeros((N, QH), F32)
    zer_0 = jnp.zeros((N, Q0), F32)
    zer16 = jnp.zeros((N, NHP), F32)

    eps = _edge_encode_project(
        edge_attr, ee_W1, r2(ee_b1), ee_W2, r2(ee_b2), Wcat, bcat)
    ep0q = eps[0:NQ]
    ep1q = eps[NQ:2 * NQ]
    ep2q = eps[2 * NQ:3 * NQ]
    epgq = eps[3 * NQ:4 * NQ]

    agg0 = _sc_gine_aggr(_stack_quarters(x), srcs4, dst, ep0q, zer_0, Q0)
    h = _gine_node(x, agg0, None, g0_W1, r2(g0_b1), g0_W2, r2(g0_b2),
                   r2(ln0_g), r2(ln0_b), has_res=False)

    agg1 = _sc_gine_aggr(_stack_quarters(h), srcs4, dst, ep1q, zer_q, QH)
    h = _gine_node(h, agg1, h, g1_W1, r2(g1_b1), g1_W2, r2(g1_b2),
                   r2(ln1_g), r2(ln1_b), has_res=True)
    agg2 = _sc_gine_aggr(_stack_quarters(h), srcs4, dst, ep2q, zer_q, QH)
    h = _gine_node(h, agg2, h, g2_W1, r2(g2_b1), g2_W2, r2(g2_b2),
                   r2(ln2_g), r2(ln2_b), has_res=True)

    xl, xr = _gat_proj(h, gat_Wl, r2(gat_bl), gat_Wr, r2(gat_br))
    xlg, xrg = _sc_gather2(xl, xr, src, dst)
    outs = _gat_edge(xlg, xrg, epgq, A, R)
    ex16, wmsgq = outs[0], outs[1:]
    den2 = _sc_scatter_den(ex16, dst, zer16)
    numer4 = _sc_scatter_q(wmsgq, dst, zer_q, QH)
    aggrg4 = _sc_scatter_q(epgq, dst, zer_q, QH)
    h = _gat_node(h, xl, xr, aggrg4, numer4, den2, A, R, r2(gat_b),
                  r2(ln3_g), r2(ln3_b))

    cv = _sc_gather_center(h, center_id)
    mu, logvar = _heads(cv, mu_W, r2(mu_b), lv_W, r2(lv_b))
    return mu, logvar


# speedup vs baseline: 5.8646x; 1.0840x over previous
import functools

import jax
import jax.numpy as jnp
from jax import lax
from jax.experimental import pallas as pl
from jax.experimental.pallas import tpu as pltpu
from jax.experimental.pallas import tpu_sc as plsc

N = 10000
E = 320000
F_IN = 128
F_E = 16
HID = 384
LAT = 64
NH = 8
HC = 48
NHP = 16
F32 = jnp.float32

NC = 2
NS = 16
NW = NC * NS
PHI = jax.lax.Precision.HIGHEST
NQ = 4
QH = HID // NQ
Q0 = F_IN // NQ

ZB = 80
NBN = N // ZB


def _edge_encode_project(ea, W1, b1, W2, b2, Wcat, bcat):
    Be = 512
    nb = E // Be

    def body(ea_ref, w1, bb1, w2, bb2, wc, bc, *outs):
        u = jnp.maximum(jnp.dot(ea_ref[...], w1[...],
                                preferred_element_type=F32) + bb1[...], 0.0)
        e = jnp.dot(u, w2[...], preferred_element_type=F32) + bb2[...]
        proj = jnp.dot(e, wc[...], preferred_element_type=F32) + bc[...]
        for q in range(NQ):
            outs[q][...] = proj[:, q * Q0:(q + 1) * Q0]
        for k in range(3 * NQ):
            outs[NQ + k][...] = proj[:, F_IN + k * QH:F_IN + (k + 1) * QH]

    full = lambda r, c: pl.BlockSpec((r, c), lambda i: (0, 0))
    eb = lambda c: pl.BlockSpec((Be, c), lambda i: (i, 0))
    sh = lambda c: jax.ShapeDtypeStruct((E, c), F32)
    return pl.pallas_call(
        body,
        grid=(nb,),
        in_specs=[
            pl.BlockSpec((Be, F_E), lambda i: (i, 0)),
            full(F_E, HID), full(1, HID), full(HID, HID), full(1, HID),
            full(HID, F_IN + 3 * HID), full(1, F_IN + 3 * HID),
        ],
        out_specs=[eb(Q0)] * NQ + [eb(QH)] * (3 * NQ),
        out_shape=[sh(Q0)] * NQ + [sh(QH)] * (3 * NQ),
    )(ea, W1, b1, W2, b2, Wcat, bcat)


def _gine_node(hin, agg4, res, W1, b1, W2, b2, g, b, has_res):
    Din = hin.shape[1]
    Dq = Din // NQ
    Bn = 1000
    nb = N // Bn
    nblk = N // Bn

    def body(*refs):
        if has_res:
            (hin_r, a0, a1, a2, a3, res_r, w1, bb1, w2, bb2, gg, bb,
             out) = refs
        else:
            (hin_r, a0, a1, a2, a3, w1, bb1, w2, bb2, gg, bb, out) = refs
        aggr = jnp.concatenate([a0[...], a1[...], a2[...], a3[...]], axis=1)
        z = hin_r[...] + aggr
        u = jnp.maximum(jnp.dot(z, w1[...], preferred_element_type=F32)
                        + bb1[...], 0.0)
        t = jnp.dot(u, w2[...], preferred_element_type=F32) + bb2[...]
        if has_res:
            t = t + res_r[...]
        m = jnp.mean(t, axis=1, keepdims=True)
        d = t - m
        v = jnp.mean(d * d, axis=1, keepdims=True)
        out[...] = d / jnp.sqrt(v + 1e-5) * gg[...] + bb[...]

    full = lambda r, c: pl.BlockSpec((r, c), lambda i: (0, 0))
    aview = lambda q: pl.BlockSpec(
        (Bn, Dq), lambda i, q=q: (i + q * nblk, 0))
    in_specs = [pl.BlockSpec((Bn, Din), lambda i: (i, 0)),
                aview(0), aview(1), aview(2), aview(3)]
    args = [hin, agg4, agg4, agg4, agg4]
    if has_res:
        in_specs.append(pl.BlockSpec((Bn, HID), lambda i: (i, 0)))
        args.append(res)
    in_specs += [full(Din, HID), full(1, HID), full(HID, HID), full(1, HID),
                 full(1, HID), full(1, HID)]
    args += [W1, b1, W2, b2, g, b]
    return pl.pallas_call(
        body,
        grid=(nb,),
        in_specs=in_specs,
        out_specs=pl.BlockSpec((Bn, HID), lambda i: (i, 0)),
        out_shape=jax.ShapeDtypeStruct((N, HID), F32),
    )(*args)


def _gat_proj(h, Wl, bl, Wr, br):
    Bn = 1000
    nb = N // Bn

    def body(h_r, wl, bbl, wr, bbr, oxl, oxr):
        hv = h_r[...]
        oxl[...] = jnp.dot(hv, wl[...], preferred_element_type=F32) + bbl[...]
        oxr[...] = jnp.dot(hv, wr[...], preferred_element_type=F32) + bbr[...]

    full = lambda r, c: pl.BlockSpec((r, c), lambda i: (0, 0))
    return pl.pallas_call(
        body,
        grid=(nb,),
        in_specs=[pl.BlockSpec((Bn, HID), lambda i: (i, 0)),
                  full(HID, HID), full(1, HID), full(HID, HID), full(1, HID)],
        out_specs=[pl.BlockSpec((Bn, HID), lambda i: (i, 0)),
                   pl.BlockSpec((Bn, HID), lambda i: (i, 0))],
        out_shape=[jax.ShapeDtypeStruct((N, HID), F32),
                   jax.ShapeDtypeStruct((N, HID), F32)],
    )(h, Wl, bl, Wr, br)


def _gat_edge(xlg, xrg, epgq, A, R):
    Be = 512
    nb = E // Be

    def body(xl_r, xr_r, e0, e1, e2, e3, a_r, r_r, oex, *ows):
        xlv = xl_r[...]
        epv = jnp.concatenate([e0[...], e1[...], e2[...], e3[...]], axis=1)
        q = xlv + xr_r[...] + epv
        ql = jnp.where(q >= 0.0, q, 0.2 * q)
        al = jnp.dot(ql, a_r[...], preferred_element_type=F32, precision=PHI)
        ex = jnp.exp(al)
        oex[...] = ex
        w = xlv * jnp.dot(ex, r_r[...], preferred_element_type=F32, precision=PHI)
        for k in range(NQ):
            ows[k][...] = w[:, k * QH:(k + 1) * QH]

    full = lambda r, c: pl.BlockSpec((r, c), lambda i: (0, 0))
    eb = lambda c: pl.BlockSpec((Be, c), lambda i: (i, 0))
    return pl.pallas_call(
        body,
        grid=(nb,),
        in_specs=[eb(HID), eb(HID)] + [eb(QH)] * NQ
        + [full(HID, NHP), full(NHP, HID)],
        out_specs=[eb(NHP)] + [eb(QH)] * NQ,
        out_shape=[jax.ShapeDtypeStruct((E, NHP), F32)]
        + [jax.ShapeDtypeStruct((E, QH), F32)] * NQ,
    )(xlg, xrg, *epgq, A, R)


def _gat_node(h, xl, xr, aggrg4, numer4, den2, A, R, gat_b, g, b):
    Bn = 1000
    nb = N // Bn
    nblk = N // Bn

    def body(h_r, xl_r, xr_r, ag0, ag1, ag2, ag3, nm0, nm1, nm2, nm3,
             d0, d1, a_r, r_r, gb, gg, bb, out):
        xlv = xl_r[...]
        den_sc = d0[...] + d1[...]
        deg = jnp.maximum(den_sc[:, 8:9], 1.0)
        lep = jnp.concatenate([ag0[...], ag1[...], ag2[...], ag3[...]],
                              axis=1) / deg
        q = xlv + xr_r[...] + lep
        ql = jnp.where(q >= 0.0, q, 0.2 * q)
        exl = jnp.exp(jnp.dot(ql, a_r[...], preferred_element_type=F32, precision=PHI))
        den_t = den_sc + exl + 1e-16
        numer = jnp.concatenate([nm0[...], nm1[...], nm2[...], nm3[...]],
                                axis=1) \
            + xlv * jnp.dot(exl, r_r[...], preferred_element_type=F32, precision=PHI)
        den_rep = jnp.dot(den_t, r_r[...], preferred_element_type=F32, precision=PHI)
        gat = numer / den_rep + gb[...]
        t = h_r[...] + gat
        m = jnp.mean(t, axis=1, keepdims=True)
        d = t - m
        v = jnp.mean(d * d, axis=1, keepdims=True)
        out[...] = d / jnp.sqrt(v + 1e-5) * gg[...] + bb[...]

    full = lambda r, c: pl.BlockSpec((r, c), lambda i: (0, 0))
    qv = lambda q: pl.BlockSpec((Bn, QH), lambda i, q=q: (i + q * nblk, 0))
    hv = lambda: pl.BlockSpec((Bn, HID), lambda i: (i, 0))
    return pl.pallas_call(
        body,
        grid=(nb,),
        in_specs=[hv(), hv(), hv(), qv(0), qv(1), qv(2), qv(3),
                  qv(0), qv(1), qv(2), qv(3),
                  pl.BlockSpec((Bn, NHP), lambda i: (i, 0)),
                  pl.BlockSpec((Bn, NHP), lambda i: (i + nblk, 0)),
                  full(HID, NHP), full(NHP, HID), full(1, HID),
                  full(1, HID), full(1, HID)],
        out_specs=pl.BlockSpec((Bn, HID), lambda i: (i, 0)),
        out_shape=jax.ShapeDtypeStruct((N, HID), F32),
    )(h, xl, xr, aggrg4, aggrg4, aggrg4, aggrg4,
      numer4, numer4, numer4, numer4, den2, den2,
      A, R, gat_b, g, b)


def _heads(cv, mu_W, mu_b, lv_W, lv_b):
    def body(cv_r, wm, bm, wv, bv, omu, olv):
        c = cv_r[...]
        omu[...] = jnp.dot(c, wm[...], preferred_element_type=F32) + bm[...]
        olv[...] = jnp.dot(c, wv[...], preferred_element_type=F32) + bv[...]

    full = lambda r, c: pl.BlockSpec((r, c), lambda i: (0, 0))
    return pl.pallas_call(
        body,
        grid=(1,),
        in_specs=[pl.BlockSpec((256, HID), lambda i: (0, 0)),
                  full(HID, LAT), full(1, LAT), full(HID, LAT), full(1, LAT)],
        out_specs=[pl.BlockSpec((256, LAT), lambda i: (0, 0)),
                   pl.BlockSpec((256, LAT), lambda i: (0, 0))],
        out_shape=[jax.ShapeDtypeStruct((256, LAT), F32),
                   jax.ShapeDtypeStruct((256, LAT), F32)],
    )(cv, mu_W, mu_b, lv_W, lv_b)


_MESH = plsc.VectorSubcoreMesh(core_axis_name="c", subcore_axis_name="s")


def _acc_rows(s, fn):
    def body(k, _):
        blk = s + k * NS

        @pl.when(blk < NBN)
        def _():
            fn(blk * ZB)
        return 0

    lax.fori_loop(0, (NBN + NS - 1) // NS, body, 0)


def _sc_gine_aggr(tab4, srcs4, dst, epq, zer, Dq):
    per = E // NS
    CH = 160
    SUB = 80
    NSUP = 5
    SROW = per // (NSUP * SUB)
    CPS = per // (NSUP * CH)
    assert NSUP * CPS * CH == per and SROW * SUB == CPS * CH
    ERB = E // SUB

    @functools.partial(
        pl.kernel, mesh=_MESH,
        compiler_params=pltpu.CompilerParams(use_tc_tiling_on_sc=False),
        out_type=jax.ShapeDtypeStruct((NQ * N, Dq), F32),
        scratch_types=[
            pltpu.VMEM((SROW, SUB), jnp.int32),
            pltpu.VMEM((SROW, SUB), jnp.int32),
            [pltpu.VMEM((CH, Dq), F32)] * 2,
            [pltpu.VMEM((CH, Dq), F32)] * 2,
            pltpu.VMEM_SHARED((N, Dq), F32),
            [pltpu.SemaphoreType.DMA] * 2,
            [pltpu.SemaphoreType.DMA] * 2,
            [pltpu.SemaphoreType.DMA] * 2,
        ],
    )
    def k(tab_r, src_r, dst_r, ep0_r, ep1_r, ep2_r, ep3_r, zer_r, out_r,
          is2d, id2d, hrs, epbs, acc, semg0, semg1, seme):
        c = lax.axis_index("c")
        s = lax.axis_index("s")
        base = s * per

        def round_(ep_r, q):
            _acc_rows(s, lambda r0: pltpu.sync_copy(zer_r.at[pl.ds(r0, ZB)],
                                                    acc.at[pl.ds(r0, ZB)]))
            plsc.subcore_barrier()

            def super_(sup, _):
                row0 = s * (per // SUB) + sup * SROW
                pltpu.sync_copy(src_r.at[pl.ds(q * ERB + row0, SROW)], is2d)
                pltpu.sync_copy(dst_r.at[pl.ds(row0, SROW)], id2d)
                sbase = base + sup * (CPS * CH)

                def stage_a(i, b):
                    e0 = sbase + i * CH
                    pltpu.async_copy(tab_r.at[is2d.at[2 * i]],
                                     hrs[b].at[pl.ds(0, SUB)], semg0[b])
                    pltpu.async_copy(tab_r.at[is2d.at[2 * i + 1]],
                                     hrs[b].at[pl.ds(SUB, SUB)], semg1[b])
                    pltpu.async_copy(ep_r.at[pl.ds(e0, CH)], epbs[b],
                                     seme[b])

                def stage_b(i, b):
                    hr, epb = hrs[b], epbs[b]
                    pltpu.make_async_copy(tab_r.at[is2d.at[0]],
                                          hr.at[pl.ds(0, SUB)],
                                          semg0[b]).wait()
                    pltpu.make_async_copy(tab_r.at[is2d.at[0]],
                                          hr.at[pl.ds(SUB, SUB)],
                                          semg1[b]).wait()
                    pltpu.make_async_copy(ep_r.at[pl.ds(0, CH)], epb,
                                          seme[b]).wait()

                    def row(r, _):
                        for j in range(Dq // 16):
                            sl = pl.ds(j * 16, 16)
                            epb[r, sl] = jnp.maximum(hr[r, sl] + epb[r, sl],
                                                     0.0)
                        return 0

                    lax.fori_loop(0, CH, row, 0)
                    pltpu.sync_copy(epb.at[pl.ds(0, SUB)],
                                    acc.at[id2d.at[2 * i]], add=True)
                    pltpu.sync_copy(epb.at[pl.ds(SUB, SUB)],
                                    acc.at[id2d.at[2 * i + 1]], add=True)

                stage_a(0, 0)

                def pair(kk, _):
                    for sub in (0, 1):
                        i = 2 * kk + sub

                        @pl.when(i + 1 < CPS)
                        def _():
                            stage_a(i + 1, 1 - sub)

                        @pl.when(i < CPS)
                        def _():
                            stage_b(i, sub)
                    return 0

                lax.fori_loop(0, (CPS + 1) // 2, pair, 0)
                return 0

            lax.fori_loop(0, NSUP, super_, 0)
            plsc.subcore_barrier()
            _acc_rows(s, lambda r0: pltpu.sync_copy(
                acc.at[pl.ds(r0, ZB)], out_r.at[pl.ds(q * N + r0, ZB)]))
            plsc.subcore_barrier()

        @pl.when(c == 0)
        def _():
            round_(ep0_r, 0)
            round_(ep1_r, 1)

        @pl.when(c == 1)
        def _():
            round_(ep2_r, 2)
            round_(ep3_r, 3)

    return k(tab4, srcs4, dst, *epq, zer)


def _sc_gather2(tabA, tabB, idxA, idxB):
    per = E // NW
    CH = 80
    nch = per // CH
    assert nch * CH == per

    @functools.partial(
        pl.kernel, mesh=_MESH,
        compiler_params=pltpu.CompilerParams(use_tc_tiling_on_sc=False),
        out_type=[jax.ShapeDtypeStruct((E, HID), F32),
                  jax.ShapeDtypeStruct((E, HID), F32)],
        scratch_types=[
            [pltpu.VMEM((CH,), jnp.int32)] * 2,
            [pltpu.VMEM((CH,), jnp.int32)] * 2,
            [pltpu.VMEM((CH, HID), F32)] * 2,
            [pltpu.VMEM((CH, HID), F32)] * 2,
            [pltpu.SemaphoreType.DMA] * 2,
            [pltpu.SemaphoreType.DMA] * 2,
        ],
    )
    def k(tA, tB, iA, iB, oA, oB, ivas, ivbs, bAs, bBs, s0s, s1s):
        c = lax.axis_index("c")
        s = lax.axis_index("s")
        wid = s * NC + c
        base = wid * per

        def stage_a(i, b):
            e0 = base + i * CH
            pltpu.sync_copy(iA.at[pl.ds(e0, CH)], ivas[b])
            pltpu.sync_copy(iB.at[pl.ds(e0, CH)], ivbs[b])
            pltpu.async_copy(tA.at[ivas[b]], bAs[b], s0s[b])
            pltpu.async_copy(tB.at[ivbs[b]], bBs[b], s1s[b])

        def stage_b(i, b):
            e0 = base + i * CH
            pltpu.make_async_copy(tA.at[ivas[b]], bAs[b], s0s[b]).wait()
            pltpu.sync_copy(bAs[b], oA.at[pl.ds(e0, CH)])
            pltpu.make_async_copy(tB.at[ivbs[b]], bBs[b], s1s[b]).wait()
            pltpu.sync_copy(bBs[b], oB.at[pl.ds(e0, CH)])

        stage_a(0, 0)

        def pair(kk, _):
            for sub in (0, 1):
                i = 2 * kk + sub

                @pl.when(i + 1 < nch)
                def _():
                    stage_a(i + 1, 1 - sub)

                @pl.when(i < nch)
                def _():
                    stage_b(i, sub)
            return 0

        lax.fori_loop(0, (nch + 1) // 2, pair, 0)

    return k(tabA, tabB, idxA, idxB)


def _sc_scatter_q(rowsq, dst, zer, Dq):
    per = E // NS
    CH = 160
    SUB = 80
    nch = per // CH

    @functools.partial(
        pl.kernel, mesh=_MESH,
        compiler_params=pltpu.CompilerParams(use_tc_tiling_on_sc=False),
        out_type=jax.ShapeDtypeStruct((NQ * N, Dq), F32),
        scratch_types=[
            [pltpu.VMEM((SUB,), jnp.int32)] * 4,
            [pltpu.VMEM((CH, Dq), F32)] * 2,
            pltpu.VMEM_SHARED((N, Dq), F32),
            [pltpu.SemaphoreType.DMA] * 2,
        ],
    )
    def k(r0_r, r1_r, r2_r, r3_r, dst_r, zer_r, out_r, ids, bufs, acc, sems):
        c = lax.axis_index("c")
        s = lax.axis_index("s")
        base = s * per

        def round_(rows_r, q):
            _acc_rows(s, lambda r0: pltpu.sync_copy(zer_r.at[pl.ds(r0, ZB)],
                                                    acc.at[pl.ds(r0, ZB)]))
            plsc.subcore_barrier()

            def stage_a(i, b):
                e0 = base + i * CH
                pltpu.sync_copy(dst_r.at[pl.ds(e0, SUB)], ids[2 * b])
                pltpu.sync_copy(dst_r.at[pl.ds(e0 + SUB, SUB)],
                                ids[2 * b + 1])
                pltpu.async_copy(rows_r.at[pl.ds(e0, CH)], bufs[b], sems[b])

            def stage_b(b):
                pltpu.make_async_copy(rows_r.at[pl.ds(0, CH)], bufs[b],
                                      sems[b]).wait()
                pltpu.sync_copy(bufs[b].at[pl.ds(0, SUB)],
                                acc.at[ids[2 * b]], add=True)
                pltpu.sync_copy(bufs[b].at[pl.ds(SUB, SUB)],
                                acc.at[ids[2 * b + 1]], add=True)

            stage_a(0, 0)

            def pair(kk, _):
                for sub in (0, 1):
                    i = 2 * kk + sub

                    @pl.when(i + 1 < nch)
                    def _():
                        stage_a(i + 1, 1 - sub)

                    @pl.when(i < nch)
                    def _():
                        stage_b(sub)
                return 0

            lax.fori_loop(0, (nch + 1) // 2, pair, 0)
            plsc.subcore_barrier()
            _acc_rows(s, lambda r0: pltpu.sync_copy(
                acc.at[pl.ds(r0, ZB)], out_r.at[pl.ds(q * N + r0, ZB)]))
            plsc.subcore_barrier()

        @pl.when(c == 0)
        def _():
            round_(r0_r, 0)
            round_(r1_r, 1)

        @pl.when(c == 1)
        def _():
            round_(r2_r, 2)
            round_(r3_r, 3)

    return k(*rowsq, dst, zer)


def _sc_scatter_den(ex16, dst, zer16):
    per = E // NW
    CH = 80
    SUB = 40
    nch = per // CH
    assert nch * CH == per

    @functools.partial(
        pl.kernel, mesh=_MESH,
        compiler_params=pltpu.CompilerParams(use_tc_tiling_on_sc=False),
        out_type=jax.ShapeDtypeStruct((2 * N, NHP), F32),
        scratch_types=[
            [pltpu.VMEM((SUB,), jnp.int32)] * 4,
            [pltpu.VMEM((CH, NHP), F32)] * 2,
            pltpu.VMEM_SHARED((N, NHP), F32),
            [pltpu.SemaphoreType.DMA] * 2,
        ],
    )
    def k(ex_r, dst_r, zer_r, out_r, ids, bufs, acc, sems):
        c = lax.axis_index("c")
        s = lax.axis_index("s")
        _acc_rows(s, lambda r0: pltpu.sync_copy(zer_r.at[pl.ds(r0, ZB)],
                                                acc.at[pl.ds(r0, ZB)]))
        plsc.subcore_barrier()
        base = (c * NS + s) * per

        def stage_a(i, b):
            e0 = base + i * CH
            pltpu.sync_copy(dst_r.at[pl.ds(e0, SUB)], ids[2 * b])
            pltpu.sync_copy(dst_r.at[pl.ds(e0 + SUB, SUB)], ids[2 * b + 1])
            pltpu.async_copy(ex_r.at[pl.ds(e0, CH)], bufs[b], sems[b])

        def stage_b(b):
            pltpu.make_async_copy(ex_r.at[pl.ds(0, CH)], bufs[b],
                                  sems[b]).wait()
            pltpu.sync_copy(bufs[b].at[pl.ds(0, SUB)], acc.at[ids[2 * b]],
                            add=True)
            pltpu.sync_copy(bufs[b].at[pl.ds(SUB, SUB)],
                            acc.at[ids[2 * b + 1]], add=True)

        stage_a(0, 0)

        def pair(kk, _):
            for sub in (0, 1):
                i = 2 * kk + sub

                @pl.when(i + 1 < nch)
                def _():
                    stage_a(i + 1, 1 - sub)

                @pl.when(i < nch)
                def _():
                    stage_b(sub)
            return 0

        lax.fori_loop(0, (nch + 1) // 2, pair, 0)
        plsc.subcore_barrier()
        _acc_rows(s, lambda r0: pltpu.sync_copy(
            acc.at[pl.ds(r0, ZB)], out_r.at[pl.ds(c * N + r0, ZB)]))

    return k(ex16, dst, zer16)


def _sc_gather_center(tab, idx):
    per = 256 // NW

    @functools.partial(
        pl.kernel, mesh=_MESH,
        compiler_params=pltpu.CompilerParams(use_tc_tiling_on_sc=False),
        out_type=jax.ShapeDtypeStruct((256, HID), F32),
        scratch_types=[
            pltpu.VMEM((per,), jnp.int32),
            pltpu.VMEM((per, HID), F32),
            pltpu.SemaphoreType.DMA,
        ],
    )
    def k(t_r, i_r, o_r, iv, buf, sem):
        c = lax.axis_index("c")
        s = lax.axis_index("s")
        wid = s * NC + c
        base = wid * per
        pltpu.sync_copy(i_r.at[pl.ds(base, per)], iv)
        pltpu.async_copy(t_r.at[iv], buf, sem).wait()
        pltpu.sync_copy(buf, o_r.at[pl.ds(base, per)])

    return k(tab, idx)


def _stack_quarters(h):
    D = h.shape[1]
    Dq = D // NQ
    return jnp.concatenate([h[:, q * Dq:(q + 1) * Dq] for q in range(NQ)],
                           axis=0)


def kernel(x, edge_attr, edge_index, center_id, ee_W1, ee_b1, ee_W2, ee_b2,
           g0_eW, g0_eb, g0_W1, g0_b1, g0_W2, g0_b2,
           g1_eW, g1_eb, g1_W1, g1_b1, g1_W2, g1_b2,
           g2_eW, g2_eb, g2_W1, g2_b1, g2_W2, g2_b2,
           gat_Wl, gat_bl, gat_Wr, gat_br, gat_We, gat_att, gat_b,
           mu_W, mu_b, lv_W, lv_b,
           ln0_g, ln0_b, ln1_g, ln1_b, ln2_g, ln2_b, ln3_g, ln3_b):
    src = edge_index[0]
    dst = edge_index[1]
    srcs4 = jnp.concatenate([src + q * N for q in range(NQ)])
    srcs4_2d = srcs4.reshape(NQ * (E // ZB), ZB)
    dst_2d = dst.reshape(E // ZB, ZB)
    r2 = lambda a: a[None, :]

    Wcat = jnp.concatenate([g0_eW, g1_eW, g2_eW, gat_We], axis=1)
    bcat = jnp.concatenate([g0_eb, g1_eb, g2_eb,
                            jnp.zeros((HID,), F32)])[None, :]
    hh = jnp.repeat(jnp.arange(NH), HC)
    A = jnp.zeros((HID, NHP), F32).at[jnp.arange(HID), hh].set(
        gat_att.reshape(-1))
    R = jnp.zeros((NHP, HID), F32).at[hh, jnp.arange(HID)].set(1.0)

    zer_q = jnp.zeros((N, QH), F32)
    zer_0 = jnp.zeros((N, Q0), F32)
    zer16 = jnp.zeros((N, NHP), F32)

    eps = _edge_encode_project(
        edge_attr, ee_W1, r2(ee_b1), ee_W2, r2(ee_b2), Wcat, bcat)
    ep0q = eps[0:NQ]
    ep1q = eps[NQ:2 * NQ]
    ep2q = eps[2 * NQ:3 * NQ]
    epgq = eps[3 * NQ:4 * NQ]

    agg0 = _sc_gine_aggr(_stack_quarters(x), srcs4_2d, dst_2d, ep0q, zer_0, Q0)
    h = _gine_node(x, agg0, None, g0_W1, r2(g0_b1), g0_W2, r2(g0_b2),
                   r2(ln0_g), r2(ln0_b), has_res=False)

    agg1 = _sc_gine_aggr(_stack_quarters(h), srcs4_2d, dst_2d, ep1q, zer_q, QH)
    h = _gine_node(h, agg1, h, g1_W1, r2(g1_b1), g1_W2, r2(g1_b2),
                   r2(ln1_g), r2(ln1_b), has_res=True)
    agg2 = _sc_gine_aggr(_stack_quarters(h), srcs4_2d, dst_2d, ep2q, zer_q, QH)
    h = _gine_node(h, agg2, h, g2_W1, r2(g2_b1), g2_W2, r2(g2_b2),
                   r2(ln2_g), r2(ln2_b), has_res=True)

    xl, xr = _gat_proj(h, gat_Wl, r2(gat_bl), gat_Wr, r2(gat_br))
    xlg, xrg = _sc_gather2(xl, xr, src, dst)
    outs = _gat_edge(xlg, xrg, epgq, A, R)
    ex16, wmsgq = outs[0], outs[1:]
    den2 = _sc_scatter_den(ex16, dst, zer16)
    numer4 = _sc_scatter_q(wmsgq, dst, zer_q, QH)
    aggrg4 = _sc_scatter_q(epgq, dst, zer_q, QH)
    h = _gat_node(h, xl, xr, aggrg4, numer4, den2, A, R, r2(gat_b),
                  r2(ln3_g), r2(ln3_b))

    cv = _sc_gather_center(h, center_id)
    mu, logvar = _heads(cv, mu_W, r2(mu_b), lv_W, r2(lv_b))
    return mu, logvar


# speedup vs baseline: 5.9193x; 1.0093x over previous
import functools

import jax
import jax.numpy as jnp
from jax import lax
from jax.experimental import pallas as pl
from jax.experimental.pallas import tpu as pltpu
from jax.experimental.pallas import tpu_sc as plsc

N = 10000
E = 320000
F_IN = 128
F_E = 16
HID = 384
LAT = 64
NH = 8
HC = 48
NHP = 16
F32 = jnp.float32

NC = 2
NS = 16
NW = NC * NS
PHI = jax.lax.Precision.HIGHEST
NQ = 4
QH = HID // NQ
Q0 = F_IN // NQ

ZB = 80
NBN = N // ZB


def _edge_encode_project(ea, W1, b1, W2, b2, Wcat, bcat):
    Be = 512
    nb = E // Be

    def body(ea_ref, w1, bb1, w2, bb2, wc, bc, *outs):
        u = jnp.maximum(jnp.dot(ea_ref[...], w1[...],
                                preferred_element_type=F32) + bb1[...], 0.0)
        e = jnp.dot(u, w2[...], preferred_element_type=F32) + bb2[...]
        proj = jnp.dot(e, wc[...], preferred_element_type=F32) + bc[...]
        for q in range(NQ):
            outs[q][...] = proj[:, q * Q0:(q + 1) * Q0]
        for k in range(3 * NQ):
            outs[NQ + k][...] = proj[:, F_IN + k * QH:F_IN + (k + 1) * QH]

    full = lambda r, c: pl.BlockSpec((r, c), lambda i: (0, 0))
    eb = lambda c: pl.BlockSpec((Be, c), lambda i: (i, 0))
    sh = lambda c: jax.ShapeDtypeStruct((E, c), F32)
    return pl.pallas_call(
        body,
        grid=(nb,),
        in_specs=[
            pl.BlockSpec((Be, F_E), lambda i: (i, 0)),
            full(F_E, HID), full(1, HID), full(HID, HID), full(1, HID),
            full(HID, F_IN + 3 * HID), full(1, F_IN + 3 * HID),
        ],
        out_specs=[eb(Q0)] * NQ + [eb(QH)] * (3 * NQ),
        out_shape=[sh(Q0)] * NQ + [sh(QH)] * (3 * NQ),
    )(ea, W1, b1, W2, b2, Wcat, bcat)


def _gine_node(hin, agg4, res, W1, b1, W2, b2, g, b, has_res):
    Din = hin.shape[1]
    Dq = Din // NQ
    Bn = 1000
    nb = N // Bn
    nblk = N // Bn

    def body(*refs):
        if has_res:
            (hin_r, a0, a1, a2, a3, res_r, w1, bb1, w2, bb2, gg, bb,
             out) = refs
        else:
            (hin_r, a0, a1, a2, a3, w1, bb1, w2, bb2, gg, bb, out) = refs
        aggr = jnp.concatenate([a0[...], a1[...], a2[...], a3[...]], axis=1)
        z = hin_r[...] + aggr
        u = jnp.maximum(jnp.dot(z, w1[...], preferred_element_type=F32)
                        + bb1[...], 0.0)
        t = jnp.dot(u, w2[...], preferred_element_type=F32) + bb2[...]
        if has_res:
            t = t + res_r[...]
        m = jnp.mean(t, axis=1, keepdims=True)
        d = t - m
        v = jnp.mean(d * d, axis=1, keepdims=True)
        out[...] = d / jnp.sqrt(v + 1e-5) * gg[...] + bb[...]

    full = lambda r, c: pl.BlockSpec((r, c), lambda i: (0, 0))
    aview = lambda q: pl.BlockSpec(
        (Bn, Dq), lambda i, q=q: (i + q * nblk, 0))
    in_specs = [pl.BlockSpec((Bn, Din), lambda i: (i, 0)),
                aview(0), aview(1), aview(2), aview(3)]
    args = [hin, agg4, agg4, agg4, agg4]
    if has_res:
        in_specs.append(pl.BlockSpec((Bn, HID), lambda i: (i, 0)))
        args.append(res)
    in_specs += [full(Din, HID), full(1, HID), full(HID, HID), full(1, HID),
                 full(1, HID), full(1, HID)]
    args += [W1, b1, W2, b2, g, b]
    return pl.pallas_call(
        body,
        grid=(nb,),
        in_specs=in_specs,
        out_specs=pl.BlockSpec((Bn, HID), lambda i: (i, 0)),
        out_shape=jax.ShapeDtypeStruct((N, HID), F32),
    )(*args)


def _gat_proj(h, Wl, bl, Wr, br):
    Bn = 1000
    nb = N // Bn

    def body(h_r, wl, bbl, wr, bbr, oxl, oxr):
        hv = h_r[...]
        oxl[...] = jnp.dot(hv, wl[...], preferred_element_type=F32) + bbl[...]
        oxr[...] = jnp.dot(hv, wr[...], preferred_element_type=F32) + bbr[...]

    full = lambda r, c: pl.BlockSpec((r, c), lambda i: (0, 0))
    return pl.pallas_call(
        body,
        grid=(nb,),
        in_specs=[pl.BlockSpec((Bn, HID), lambda i: (i, 0)),
                  full(HID, HID), full(1, HID), full(HID, HID), full(1, HID)],
        out_specs=[pl.BlockSpec((Bn, HID), lambda i: (i, 0)),
                   pl.BlockSpec((Bn, HID), lambda i: (i, 0))],
        out_shape=[jax.ShapeDtypeStruct((N, HID), F32),
                   jax.ShapeDtypeStruct((N, HID), F32)],
    )(h, Wl, bl, Wr, br)


def _gat_edge(xlg, xrg, epgq, A, R):
    Be = 512
    nb = E // Be

    def body(xl_r, xr_r, e0, e1, e2, e3, a_r, r_r, oex, *ows):
        xlv = xl_r[...]
        epv = jnp.concatenate([e0[...], e1[...], e2[...], e3[...]], axis=1)
        q = xlv + xr_r[...] + epv
        ql = jnp.where(q >= 0.0, q, 0.2 * q)
        al = jnp.dot(ql, a_r[...], preferred_element_type=F32, precision=PHI)
        ex = jnp.exp(al)
        oex[...] = ex
        w = xlv * jnp.dot(ex, r_r[...], preferred_element_type=F32, precision=PHI)
        for k in range(NQ):
            ows[k][...] = w[:, k * QH:(k + 1) * QH]

    full = lambda r, c: pl.BlockSpec((r, c), lambda i: (0, 0))
    eb = lambda c: pl.BlockSpec((Be, c), lambda i: (i, 0))
    return pl.pallas_call(
        body,
        grid=(nb,),
        in_specs=[eb(HID), eb(HID)] + [eb(QH)] * NQ
        + [full(HID, NHP), full(NHP, HID)],
        out_specs=[eb(NHP)] + [eb(QH)] * NQ,
        out_shape=[jax.ShapeDtypeStruct((E, NHP), F32)]
        + [jax.ShapeDtypeStruct((E, QH), F32)] * NQ,
    )(xlg, xrg, *epgq, A, R)


def _gat_node(h, xl, xr, aggrg4, numer4, den2, A, R, gat_b, g, b):
    Bn = 1000
    nb = N // Bn
    nblk = N // Bn

    def body(h_r, xl_r, xr_r, ag0, ag1, ag2, ag3, nm0, nm1, nm2, nm3,
             d0, d1, a_r, r_r, gb, gg, bb, out):
        xlv = xl_r[...]
        den_sc = d0[...] + d1[...]
        deg = jnp.maximum(den_sc[:, 8:9], 1.0)
        lep = jnp.concatenate([ag0[...], ag1[...], ag2[...], ag3[...]],
                              axis=1) / deg
        q = xlv + xr_r[...] + lep
        ql = jnp.where(q >= 0.0, q, 0.2 * q)
        exl = jnp.exp(jnp.dot(ql, a_r[...], preferred_element_type=F32, precision=PHI))
        den_t = den_sc + exl + 1e-16
        numer = jnp.concatenate([nm0[...], nm1[...], nm2[...], nm3[...]],
                                axis=1) \
            + xlv * jnp.dot(exl, r_r[...], preferred_element_type=F32, precision=PHI)
        den_rep = jnp.dot(den_t, r_r[...], preferred_element_type=F32, precision=PHI)
        gat = numer / den_rep + gb[...]
        t = h_r[...] + gat
        m = jnp.mean(t, axis=1, keepdims=True)
        d = t - m
        v = jnp.mean(d * d, axis=1, keepdims=True)
        out[...] = d / jnp.sqrt(v + 1e-5) * gg[...] + bb[...]

    full = lambda r, c: pl.BlockSpec((r, c), lambda i: (0, 0))
    qv = lambda q: pl.BlockSpec((Bn, QH), lambda i, q=q: (i + q * nblk, 0))
    hv = lambda: pl.BlockSpec((Bn, HID), lambda i: (i, 0))
    return pl.pallas_call(
        body,
        grid=(nb,),
        in_specs=[hv(), hv(), hv(), qv(0), qv(1), qv(2), qv(3),
                  qv(0), qv(1), qv(2), qv(3),
                  pl.BlockSpec((Bn, NHP), lambda i: (i, 0)),
                  pl.BlockSpec((Bn, NHP), lambda i: (i + nblk, 0)),
                  full(HID, NHP), full(NHP, HID), full(1, HID),
                  full(1, HID), full(1, HID)],
        out_specs=pl.BlockSpec((Bn, HID), lambda i: (i, 0)),
        out_shape=jax.ShapeDtypeStruct((N, HID), F32),
    )(h, xl, xr, aggrg4, aggrg4, aggrg4, aggrg4,
      numer4, numer4, numer4, numer4, den2, den2,
      A, R, gat_b, g, b)


def _heads(cv, mu_W, mu_b, lv_W, lv_b):
    def body(cv_r, wm, bm, wv, bv, omu, olv):
        c = cv_r[...]
        omu[...] = jnp.dot(c, wm[...], preferred_element_type=F32) + bm[...]
        olv[...] = jnp.dot(c, wv[...], preferred_element_type=F32) + bv[...]

    full = lambda r, c: pl.BlockSpec((r, c), lambda i: (0, 0))
    return pl.pallas_call(
        body,
        grid=(1,),
        in_specs=[pl.BlockSpec((256, HID), lambda i: (0, 0)),
                  full(HID, LAT), full(1, LAT), full(HID, LAT), full(1, LAT)],
        out_specs=[pl.BlockSpec((256, LAT), lambda i: (0, 0)),
                   pl.BlockSpec((256, LAT), lambda i: (0, 0))],
        out_shape=[jax.ShapeDtypeStruct((256, LAT), F32),
                   jax.ShapeDtypeStruct((256, LAT), F32)],
    )(cv, mu_W, mu_b, lv_W, lv_b)


_MESH = plsc.VectorSubcoreMesh(core_axis_name="c", subcore_axis_name="s")


def _acc_rows(s, fn):
    def body(k, _):
        blk = s + k * NS

        @pl.when(blk < NBN)
        def _():
            fn(blk * ZB)
        return 0

    lax.fori_loop(0, (NBN + NS - 1) // NS, body, 0)


def _sc_gine_aggr(tab4, srcs4, dst, epq, zer, Dq):
    per = E // NS
    CH = 160
    SUB = 80
    NSUP = 5
    SROW = per // (NSUP * SUB)
    CPS = per // (NSUP * CH)
    assert NSUP * CPS * CH == per and SROW * SUB == CPS * CH
    ERB = E // SUB

    @functools.partial(
        pl.kernel, mesh=_MESH,
        compiler_params=pltpu.CompilerParams(use_tc_tiling_on_sc=False),
        out_type=jax.ShapeDtypeStruct((NQ * N, Dq), F32),
        scratch_types=[
            pltpu.VMEM((SROW, SUB), jnp.int32),
            pltpu.VMEM((SROW, SUB), jnp.int32),
            [pltpu.VMEM((CH, Dq), F32)] * 2,
            [pltpu.VMEM((CH, Dq), F32)] * 2,
            pltpu.VMEM_SHARED((N, Dq), F32),
            [pltpu.SemaphoreType.DMA] * 2,
            [pltpu.SemaphoreType.DMA] * 2,
            [pltpu.SemaphoreType.DMA] * 2,
        ],
    )
    def k(tab_r, src_r, dst_r, ep0_r, ep1_r, ep2_r, ep3_r, zer_r, out_r,
          is2d, id2d, hrs, epbs, acc, semg0, semg1, seme):
        c = lax.axis_index("c")
        s = lax.axis_index("s")
        base = s * per

        def round_(ep_r, q):
            _acc_rows(s, lambda r0: pltpu.sync_copy(zer_r.at[pl.ds(r0, ZB)],
                                                    acc.at[pl.ds(r0, ZB)]))
            plsc.subcore_barrier()

            def super_(sup, _):
                row0 = s * (per // SUB) + sup * SROW
                pltpu.sync_copy(src_r.at[pl.ds(q * ERB + row0, SROW)], is2d)
                pltpu.sync_copy(dst_r.at[pl.ds(row0, SROW)], id2d)
                sbase = base + sup * (CPS * CH)

                def stage_a(i, b):
                    e0 = sbase + i * CH
                    pltpu.async_copy(tab_r.at[is2d.at[2 * i]],
                                     hrs[b].at[pl.ds(0, SUB)], semg0[b])
                    pltpu.async_copy(tab_r.at[is2d.at[2 * i + 1]],
                                     hrs[b].at[pl.ds(SUB, SUB)], semg1[b])
                    pltpu.async_copy(ep_r.at[pl.ds(e0, CH)], epbs[b],
                                     seme[b])

                def stage_b(i, b):
                    hr, epb = hrs[b], epbs[b]
                    pltpu.make_async_copy(tab_r.at[is2d.at[0]],
                                          hr.at[pl.ds(0, SUB)],
                                          semg0[b]).wait()
                    pltpu.make_async_copy(tab_r.at[is2d.at[0]],
                                          hr.at[pl.ds(SUB, SUB)],
                                          semg1[b]).wait()
                    pltpu.make_async_copy(ep_r.at[pl.ds(0, CH)], epb,
                                          seme[b]).wait()

                    def row(r, _):
                        for j in range(Dq // 16):
                            sl = pl.ds(j * 16, 16)
                            epb[r, sl] = jnp.maximum(hr[r, sl] + epb[r, sl],
                                                     0.0)
                        return 0

                    lax.fori_loop(0, CH, row, 0)
                    pltpu.sync_copy(epb.at[pl.ds(0, SUB)],
                                    acc.at[id2d.at[2 * i]], add=True)
                    pltpu.sync_copy(epb.at[pl.ds(SUB, SUB)],
                                    acc.at[id2d.at[2 * i + 1]], add=True)

                stage_a(0, 0)

                def pair(kk, _):
                    for sub in (0, 1):
                        i = 2 * kk + sub

                        @pl.when(i + 1 < CPS)
                        def _():
                            stage_a(i + 1, 1 - sub)

                        @pl.when(i < CPS)
                        def _():
                            stage_b(i, sub)
                    return 0

                lax.fori_loop(0, (CPS + 1) // 2, pair, 0)
                return 0

            lax.fori_loop(0, NSUP, super_, 0)
            plsc.subcore_barrier()
            _acc_rows(s, lambda r0: pltpu.sync_copy(
                acc.at[pl.ds(r0, ZB)], out_r.at[pl.ds(q * N + r0, ZB)]))
            plsc.subcore_barrier()

        @pl.when(c == 0)
        def _():
            round_(ep0_r, 0)
            round_(ep1_r, 1)

        @pl.when(c == 1)
        def _():
            round_(ep2_r, 2)
            round_(ep3_r, 3)

    return k(tab4, srcs4, dst, *epq, zer)


def _sc_gather2(tabA, tabB, idxA, idxB):
    per = E // NW
    CH = 80
    nch = per // CH
    GSR = 25
    assert nch * CH == per and nch % GSR == 0

    @functools.partial(
        pl.kernel, mesh=_MESH,
        compiler_params=pltpu.CompilerParams(use_tc_tiling_on_sc=False),
        out_type=[jax.ShapeDtypeStruct((E, HID), F32),
                  jax.ShapeDtypeStruct((E, HID), F32)],
        scratch_types=[
            pltpu.VMEM((GSR, CH), jnp.int32),
            pltpu.VMEM((GSR, CH), jnp.int32),
            [pltpu.VMEM((CH, HID), F32)] * 2,
            [pltpu.VMEM((CH, HID), F32)] * 2,
            [pltpu.SemaphoreType.DMA] * 2,
            [pltpu.SemaphoreType.DMA] * 2,
        ],
    )
    def k(tA, tB, iA, iB, oA, oB, ivas, ivbs, bAs, bBs, s0s, s1s):
        c = lax.axis_index("c")
        s = lax.axis_index("s")
        wid = s * NC + c
        base = wid * per
        row_base = wid * (per // CH)

        def super_(sup, _):
            pltpu.sync_copy(iA.at[pl.ds(row_base + sup * GSR, GSR)], ivas)
            pltpu.sync_copy(iB.at[pl.ds(row_base + sup * GSR, GSR)], ivbs)
            sbase = base + sup * GSR * CH

            def stage_a(i, b):
                pltpu.async_copy(tA.at[ivas.at[i]], bAs[b], s0s[b])
                pltpu.async_copy(tB.at[ivbs.at[i]], bBs[b], s1s[b])

            def stage_b(i, b):
                e0 = sbase + i * CH
                pltpu.make_async_copy(tA.at[ivas.at[0]], bAs[b],
                                      s0s[b]).wait()
                pltpu.sync_copy(bAs[b], oA.at[pl.ds(e0, CH)])
                pltpu.make_async_copy(tB.at[ivbs.at[0]], bBs[b],
                                      s1s[b]).wait()
                pltpu.sync_copy(bBs[b], oB.at[pl.ds(e0, CH)])

            stage_a(0, 0)

            def pair(kk, _):
                for sub in (0, 1):
                    i = 2 * kk + sub

                    @pl.when(i + 1 < GSR)
                    def _():
                        stage_a(i + 1, 1 - sub)

                    @pl.when(i < GSR)
                    def _():
                        stage_b(i, sub)
                return 0

            lax.fori_loop(0, (GSR + 1) // 2, pair, 0)
            return 0

        lax.fori_loop(0, nch // GSR, super_, 0)

    return k(tabA, tabB, idxA, idxB)


def _sc_scatter_q(rowsq, dst, zer, Dq):
    per = E // NS
    CH = 160
    SUB = 80
    NSUP = 5
    SROW = per // (NSUP * SUB)
    CPS = per // (NSUP * CH)
    assert NSUP * CPS * CH == per and SROW * SUB == CPS * CH

    @functools.partial(
        pl.kernel, mesh=_MESH,
        compiler_params=pltpu.CompilerParams(use_tc_tiling_on_sc=False),
        out_type=jax.ShapeDtypeStruct((NQ * N, Dq), F32),
        scratch_types=[
            pltpu.VMEM((SROW, SUB), jnp.int32),
            [pltpu.VMEM((CH, Dq), F32)] * 2,
            pltpu.VMEM_SHARED((N, Dq), F32),
            [pltpu.SemaphoreType.DMA] * 2,
        ],
    )
    def k(r0_r, r1_r, r2_r, r3_r, dst_r, zer_r, out_r, id2d, bufs, acc,
          sems):
        c = lax.axis_index("c")
        s = lax.axis_index("s")
        base = s * per

        def round_(rows_r, q):
            _acc_rows(s, lambda r0: pltpu.sync_copy(zer_r.at[pl.ds(r0, ZB)],
                                                    acc.at[pl.ds(r0, ZB)]))
            plsc.subcore_barrier()

            def super_(sup, _):
                row0 = s * (per // SUB) + sup * SROW
                pltpu.sync_copy(dst_r.at[pl.ds(row0, SROW)], id2d)
                sbase = base + sup * (CPS * CH)

                def stage_a(i, b):
                    e0 = sbase + i * CH
                    pltpu.async_copy(rows_r.at[pl.ds(e0, CH)], bufs[b],
                                     sems[b])

                def stage_b(i, b):
                    pltpu.make_async_copy(rows_r.at[pl.ds(0, CH)], bufs[b],
                                          sems[b]).wait()
                    pltpu.sync_copy(bufs[b].at[pl.ds(0, SUB)],
                                    acc.at[id2d.at[2 * i]], add=True)
                    pltpu.sync_copy(bufs[b].at[pl.ds(SUB, SUB)],
                                    acc.at[id2d.at[2 * i + 1]], add=True)

                stage_a(0, 0)

                def pair(kk, _):
                    for sub in (0, 1):
                        i = 2 * kk + sub

                        @pl.when(i + 1 < CPS)
                        def _():
                            stage_a(i + 1, 1 - sub)

                        @pl.when(i < CPS)
                        def _():
                            stage_b(i, sub)
                    return 0

                lax.fori_loop(0, (CPS + 1) // 2, pair, 0)
                return 0

            lax.fori_loop(0, NSUP, super_, 0)
            plsc.subcore_barrier()
            _acc_rows(s, lambda r0: pltpu.sync_copy(
                acc.at[pl.ds(r0, ZB)], out_r.at[pl.ds(q * N + r0, ZB)]))
            plsc.subcore_barrier()

        @pl.when(c == 0)
        def _():
            round_(r0_r, 0)
            round_(r1_r, 1)

        @pl.when(c == 1)
        def _():
            round_(r2_r, 2)
            round_(r3_r, 3)

    return k(*rowsq, dst, zer)


def _sc_scatter_den(ex16, dst, zer16):
    per = E // NW
    CH = 80
    SUB = 40
    nch = per // CH
    assert nch * CH == per

    @functools.partial(
        pl.kernel, mesh=_MESH,
        compiler_params=pltpu.CompilerParams(use_tc_tiling_on_sc=False),
        out_type=jax.ShapeDtypeStruct((2 * N, NHP), F32),
        scratch_types=[
            [pltpu.VMEM((SUB,), jnp.int32)] * 4,
            [pltpu.VMEM((CH, NHP), F32)] * 2,
            pltpu.VMEM_SHARED((N, NHP), F32),
            [pltpu.SemaphoreType.DMA] * 2,
        ],
    )
    def k(ex_r, dst_r, zer_r, out_r, ids, bufs, acc, sems):
        c = lax.axis_index("c")
        s = lax.axis_index("s")
        _acc_rows(s, lambda r0: pltpu.sync_copy(zer_r.at[pl.ds(r0, ZB)],
                                                acc.at[pl.ds(r0, ZB)]))
        plsc.subcore_barrier()
        base = (c * NS + s) * per

        def stage_a(i, b):
            e0 = base + i * CH
            pltpu.sync_copy(dst_r.at[pl.ds(e0, SUB)], ids[2 * b])
            pltpu.sync_copy(dst_r.at[pl.ds(e0 + SUB, SUB)], ids[2 * b + 1])
            pltpu.async_copy(ex_r.at[pl.ds(e0, CH)], bufs[b], sems[b])

        def stage_b(b):
            pltpu.make_async_copy(ex_r.at[pl.ds(0, CH)], bufs[b],
                                  sems[b]).wait()
            pltpu.sync_copy(bufs[b].at[pl.ds(0, SUB)], acc.at[ids[2 * b]],
                            add=True)
            pltpu.sync_copy(bufs[b].at[pl.ds(SUB, SUB)],
                            acc.at[ids[2 * b + 1]], add=True)

        stage_a(0, 0)

        def pair(kk, _):
            for sub in (0, 1):
                i = 2 * kk + sub

                @pl.when(i + 1 < nch)
                def _():
                    stage_a(i + 1, 1 - sub)

                @pl.when(i < nch)
                def _():
                    stage_b(sub)
            return 0

        lax.fori_loop(0, (nch + 1) // 2, pair, 0)
        plsc.subcore_barrier()
        _acc_rows(s, lambda r0: pltpu.sync_copy(
            acc.at[pl.ds(r0, ZB)], out_r.at[pl.ds(c * N + r0, ZB)]))

    return k(ex16, dst, zer16)


def _sc_gather_center(tab, idx):
    per = 256 // NW

    @functools.partial(
        pl.kernel, mesh=_MESH,
        compiler_params=pltpu.CompilerParams(use_tc_tiling_on_sc=False),
        out_type=jax.ShapeDtypeStruct((256, HID), F32),
        scratch_types=[
            pltpu.VMEM((per,), jnp.int32),
            pltpu.VMEM((per, HID), F32),
            pltpu.SemaphoreType.DMA,
        ],
    )
    def k(t_r, i_r, o_r, iv, buf, sem):
        c = lax.axis_index("c")
        s = lax.axis_index("s")
        wid = s * NC + c
        base = wid * per
        pltpu.sync_copy(i_r.at[pl.ds(base, per)], iv)
        pltpu.async_copy(t_r.at[iv], buf, sem).wait()
        pltpu.sync_copy(buf, o_r.at[pl.ds(base, per)])

    return k(tab, idx)


def _stack_quarters(h):
    D = h.shape[1]
    Dq = D // NQ
    return jnp.concatenate([h[:, q * Dq:(q + 1) * Dq] for q in range(NQ)],
                           axis=0)


def kernel(x, edge_attr, edge_index, center_id, ee_W1, ee_b1, ee_W2, ee_b2,
           g0_eW, g0_eb, g0_W1, g0_b1, g0_W2, g0_b2,
           g1_eW, g1_eb, g1_W1, g1_b1, g1_W2, g1_b2,
           g2_eW, g2_eb, g2_W1, g2_b1, g2_W2, g2_b2,
           gat_Wl, gat_bl, gat_Wr, gat_br, gat_We, gat_att, gat_b,
           mu_W, mu_b, lv_W, lv_b,
           ln0_g, ln0_b, ln1_g, ln1_b, ln2_g, ln2_b, ln3_g, ln3_b):
    src = edge_index[0]
    dst = edge_index[1]
    srcs4 = jnp.concatenate([src + q * N for q in range(NQ)])
    srcs4_2d = srcs4.reshape(NQ * (E // ZB), ZB)
    dst_2d = dst.reshape(E // ZB, ZB)
    src_2d = src.reshape(E // ZB, ZB)
    r2 = lambda a: a[None, :]

    Wcat = jnp.concatenate([g0_eW, g1_eW, g2_eW, gat_We], axis=1)
    bcat = jnp.concatenate([g0_eb, g1_eb, g2_eb,
                            jnp.zeros((HID,), F32)])[None, :]
    hh = jnp.repeat(jnp.arange(NH), HC)
    A = jnp.zeros((HID, NHP), F32).at[jnp.arange(HID), hh].set(
        gat_att.reshape(-1))
    R = jnp.zeros((NHP, HID), F32).at[hh, jnp.arange(HID)].set(1.0)

    zer_q = jnp.zeros((N, QH), F32)
    zer_0 = jnp.zeros((N, Q0), F32)
    zer16 = jnp.zeros((N, NHP), F32)

    eps = _edge_encode_project(
        edge_attr, ee_W1, r2(ee_b1), ee_W2, r2(ee_b2), Wcat, bcat)
    ep0q = eps[0:NQ]
    ep1q = eps[NQ:2 * NQ]
    ep2q = eps[2 * NQ:3 * NQ]
    epgq = eps[3 * NQ:4 * NQ]

    agg0 = _sc_gine_aggr(_stack_quarters(x), srcs4_2d, dst_2d, ep0q, zer_0, Q0)
    h = _gine_node(x, agg0, None, g0_W1, r2(g0_b1), g0_W2, r2(g0_b2),
                   r2(ln0_g), r2(ln0_b), has_res=False)

    agg1 = _sc_gine_aggr(_stack_quarters(h), srcs4_2d, dst_2d, ep1q, zer_q, QH)
    h = _gine_node(h, agg1, h, g1_W1, r2(g1_b1), g1_W2, r2(g1_b2),
                   r2(ln1_g), r2(ln1_b), has_res=True)
    agg2 = _sc_gine_aggr(_stack_quarters(h), srcs4_2d, dst_2d, ep2q, zer_q, QH)
    h = _gine_node(h, agg2, h, g2_W1, r2(g2_b1), g2_W2, r2(g2_b2),
                   r2(ln2_g), r2(ln2_b), has_res=True)

    xl, xr = _gat_proj(h, gat_Wl, r2(gat_bl), gat_Wr, r2(gat_br))
    xlg, xrg = _sc_gather2(xl, xr, src_2d, dst_2d)
    outs = _gat_edge(xlg, xrg, epgq, A, R)
    ex16, wmsgq = outs[0], outs[1:]
    den2 = _sc_scatter_den(ex16, dst, zer16)
    numer4 = _sc_scatter_q(wmsgq, dst_2d, zer_q, QH)
    aggrg4 = _sc_scatter_q(epgq, dst_2d, zer_q, QH)
    h = _gat_node(h, xl, xr, aggrg4, numer4, den2, A, R, r2(gat_b),
                  r2(ln3_g), r2(ln3_b))

    cv = _sc_gather_center(h, center_id)
    mu, logvar = _heads(cv, mu_W, r2(mu_b), lv_W, r2(lv_b))
    return mu, logvar
